# Initial kernel scaffold; baseline (speedup 1.0000x reference)
import jax
import jax.numpy as jnp
from jax.experimental import pallas as pl


def kernel(inputs, W0, b0, W1, b1, W2, b2, W3, b3, W4, b4):
    raise NotImplementedError("write your pallas kernel here")


# speedup vs baseline: 650.7094x; 650.7094x over previous
import functools

import jax
import jax.numpy as jnp
import numpy as np
from jax.experimental import pallas as pl
from jax.experimental.pallas import tpu as pltpu

NUM_MODELS = 64
BATCH = 8192
HID = 64
INF = 6
OUTF = 3
NPASS = 4
TILE = 128
GRID = BATCH // TILE
SEL_TILE = 1024


def _sel_kernel(x_ref, idx_ref, logit_ref, prob_ref):
    x0 = x_ref[:, 0:1]
    x2 = x_ref[:, 2:3]
    ang = jnp.arctan2(x2, x0)
    ang = jnp.mod(ang + 2 * np.pi, 2 * np.pi) / (2 * np.pi) * NUM_MODELS
    idx_ref[...] = jnp.floor(ang).astype(jnp.int32)
    logit_ref[...] = jnp.ones_like(logit_ref)
    prob_ref[...] = jnp.full_like(prob_ref, 1.0 / NUM_MODELS)


def _selection(inputs):
    nblk = BATCH // SEL_TILE
    idx, logits, probs = pl.pallas_call(
        _sel_kernel,
        grid=(nblk,),
        in_specs=[pl.BlockSpec((SEL_TILE, INF), lambda i: (i, 0))],
        out_specs=[
            pl.BlockSpec((SEL_TILE, 1), lambda i: (i, 0)),
            pl.BlockSpec((SEL_TILE, NUM_MODELS), lambda i: (i, 0)),
            pl.BlockSpec((SEL_TILE, NUM_MODELS), lambda i: (i, 0)),
        ],
        out_shape=[
            jax.ShapeDtypeStruct((BATCH, 1), jnp.int32),
            jax.ShapeDtypeStruct((BATCH, NUM_MODELS), jnp.float32),
            jax.ShapeDtypeStruct((BATCH, NUM_MODELS), jnp.float32),
        ],
    )(inputs)
    return idx[:, 0], logits, probs


def _mlp_kernel(ids_sm, x_ref, idc_ref, w0, b0, w1, b1, w2, b2, w3, b3, w4,
                b4, out_ref):
    g = pl.program_id(0)
    e_lo = ids_sm[g * TILE]
    e_hi = ids_sm[g * TILE + TILE - 1]
    x = x_ref[...]
    idcol = idc_ref[...]
    idcol4 = jnp.concatenate([idcol] * NPASS, axis=0)

    row = jax.lax.broadcasted_iota(jnp.int32, (NPASS * TILE, HID), 0)
    col = jax.lax.broadcasted_iota(jnp.int32, (NPASS * TILE, HID), 1)
    colmask = (col < (HID // NPASS) * (row // TILE + 1)).astype(jnp.float32)

    def body(e, acc):
        y0 = jax.lax.dot_general(x, w0[e], (((1,), (1,)), ((), ())),
                                 preferred_element_type=jnp.float32)
        y0 = jnp.maximum(y0 + b0[e], 0.0)
        xs = jnp.concatenate([y0] * NPASS, axis=0) * colmask
        for w, b in ((w1, b1), (w2, b2), (w3, b3)):
            xs = jax.lax.dot_general(xs, w[e], (((1,), (1,)), ((), ())),
                                     preferred_element_type=jnp.float32)
            xs = jnp.maximum(xs + b[e], 0.0) * colmask
        y = jax.lax.dot_general(xs, w4[e], (((1,), (1,)), ((), ())),
                                preferred_element_type=jnp.float32) + b4[e]
        return jnp.where(idcol4 == e, y, acc)

    acc = jnp.zeros((NPASS * TILE, OUTF), jnp.float32)
    acc = jax.lax.fori_loop(e_lo, e_hi + 1, body, acc)
    out_ref[...] = acc.reshape(NPASS, TILE, OUTF)


def _full(shape):
    zeros = tuple(0 for _ in shape)
    return pl.BlockSpec(shape, lambda g, ids, z=zeros: z)


def _grouped_mlp(ids_sorted, x_sorted, Ws, bs, interpret=False):
    idc = ids_sorted.reshape(BATCH, 1)
    b0r, b1r, b2r, b3r = (b.reshape(NUM_MODELS, 1, HID) for b in bs[:4])
    b4r = bs[4].reshape(NUM_MODELS, 1, OUTF)
    grid_spec = pltpu.PrefetchScalarGridSpec(
        num_scalar_prefetch=1,
        grid=(GRID,),
        in_specs=[
            pl.BlockSpec((TILE, INF), lambda g, ids: (g, 0)),
            pl.BlockSpec((TILE, 1), lambda g, ids: (g, 0)),
            _full((NUM_MODELS, HID, INF)),
            _full((NUM_MODELS, 1, HID)),
            _full((NUM_MODELS, HID, HID)),
            _full((NUM_MODELS, 1, HID)),
            _full((NUM_MODELS, HID, HID)),
            _full((NUM_MODELS, 1, HID)),
            _full((NUM_MODELS, HID, HID)),
            _full((NUM_MODELS, 1, HID)),
            _full((NUM_MODELS, OUTF, HID)),
            _full((NUM_MODELS, 1, OUTF)),
        ],
        out_specs=pl.BlockSpec((NPASS, TILE, OUTF), lambda g, ids: (0, g, 0)),
    )
    return pl.pallas_call(
        _mlp_kernel,
        grid_spec=grid_spec,
        out_shape=jax.ShapeDtypeStruct((NPASS, BATCH, OUTF), jnp.float32),
        interpret=interpret,
    )(ids_sorted, x_sorted, idc, Ws[0], b0r, Ws[1], b1r, Ws[2], b2r, Ws[3],
      b3r, Ws[4], b4r)


def kernel(inputs, W0, b0, W1, b1, W2, b2, W3, b3, W4, b4):
    idx, logits, probs = _selection(inputs)
    perm = jnp.argsort(idx)
    x_sorted = inputs[perm]
    ids_sorted = idx[perm]
    y_sorted = _grouped_mlp(ids_sorted, x_sorted,
                            (W0, W1, W2, W3, W4), (b0, b1, b2, b3, b4))
    inv = jnp.argsort(perm)
    model_outputs = y_sorted[:, inv, :]
    return (model_outputs, idx, logits, probs)


# speedup vs baseline: 768.1981x; 1.1806x over previous
import functools

import jax
import jax.numpy as jnp
import numpy as np
from jax.experimental import pallas as pl
from jax.experimental.pallas import tpu as pltpu
from jax.experimental.pallas import tpu_sc as plsc

NUM_MODELS = 64
BATCH = 8192
HID = 64
INF = 6
OUTF = 3
NPASS = 4
ROWW = 16
TILE = 128
GRID = BATCH // TILE
SEL_TILE = 1024
SC_WORKERS = 32
SC_CHUNK = 128


def _sel_kernel(x_ref, idx_ref, logit_ref, prob_ref):
    x0 = x_ref[:, 0:1]
    x2 = x_ref[:, 2:3]
    ang = jnp.arctan2(x2, x0)
    ang = jnp.mod(ang + 2 * np.pi, 2 * np.pi) / (2 * np.pi) * NUM_MODELS
    idx_ref[...] = jnp.floor(ang).astype(jnp.int32)
    logit_ref[...] = jnp.ones_like(logit_ref)
    prob_ref[...] = jnp.full_like(prob_ref, 1.0 / NUM_MODELS)


def _selection(inputs):
    nblk = BATCH // SEL_TILE
    idx, logits, probs = pl.pallas_call(
        _sel_kernel,
        grid=(nblk,),
        in_specs=[pl.BlockSpec((SEL_TILE, INF), lambda i: (i, 0))],
        out_specs=[
            pl.BlockSpec((SEL_TILE, 1), lambda i: (i, 0)),
            pl.BlockSpec((SEL_TILE, NUM_MODELS), lambda i: (i, 0)),
            pl.BlockSpec((SEL_TILE, NUM_MODELS), lambda i: (i, 0)),
        ],
        out_shape=[
            jax.ShapeDtypeStruct((BATCH, 1), jnp.int32),
            jax.ShapeDtypeStruct((BATCH, NUM_MODELS), jnp.float32),
            jax.ShapeDtypeStruct((BATCH, NUM_MODELS), jnp.float32),
        ],
    )(inputs)
    return idx[:, 0], logits, probs


def _sc_row_gather(table, idx2d):
    nrows, ncols = table.shape
    b_per_w = nrows // SC_WORKERS
    nchunk = b_per_w // SC_CHUNK
    mesh = plsc.VectorSubcoreMesh(core_axis_name="c", subcore_axis_name="s")

    @functools.partial(
        pl.kernel, mesh=mesh,
        out_type=jax.ShapeDtypeStruct((nrows, ncols), jnp.float32),
        compiler_params=pltpu.CompilerParams(use_tc_tiling_on_sc=False),
        scratch_types=[
            pltpu.VMEM((nchunk, SC_CHUNK), jnp.int32),
            pltpu.VMEM((b_per_w, ncols), jnp.float32),
            pltpu.SemaphoreType.DMA,
        ],
    )
    def k(table_hbm, idx_hbm, out_hbm, idx_v, rows_v, sem):
        wid = jax.lax.axis_index("s") * 2 + jax.lax.axis_index("c")
        base = wid * b_per_w
        pltpu.sync_copy(idx_hbm.at[pl.ds(wid * nchunk, nchunk)], idx_v)
        copies = [
            pltpu.async_copy(table_hbm.at[idx_v.at[j]],
                             rows_v.at[pl.ds(j * SC_CHUNK, SC_CHUNK)], sem)
            for j in range(nchunk)
        ]
        for c in copies:
            c.wait()
        pltpu.sync_copy(rows_v, out_hbm.at[pl.ds(base, b_per_w)])

    return k(table, idx2d)


def _mlp_kernel(ids_sm, x_ref, w0, b0, w1, b1, w2, b2, w3, b3, w4, b4,
                out_ref):
    g = pl.program_id(0)
    e_lo = ids_sm[g * TILE]
    e_hi = ids_sm[g * TILE + TILE - 1]
    x = x_ref[:, :INF]
    idcol = x_ref[:, INF:INF + 1].astype(jnp.int32)
    idcol4 = jnp.concatenate([idcol] * NPASS, axis=0)

    row = jax.lax.broadcasted_iota(jnp.int32, (NPASS * TILE, HID), 0)
    col = jax.lax.broadcasted_iota(jnp.int32, (NPASS * TILE, HID), 1)
    colmask = (col < (HID // NPASS) * (row // TILE + 1)).astype(jnp.float32)

    def body(e, acc):
        y0 = jax.lax.dot_general(x, w0[e], (((1,), (1,)), ((), ())),
                                 preferred_element_type=jnp.float32)
        y0 = jnp.maximum(y0 + b0[e], 0.0)
        xs = jnp.concatenate([y0] * NPASS, axis=0) * colmask
        for w, b in ((w1, b1), (w2, b2), (w3, b3)):
            xs = jax.lax.dot_general(xs, w[e], (((1,), (1,)), ((), ())),
                                     preferred_element_type=jnp.float32)
            xs = jnp.maximum(xs + b[e], 0.0) * colmask
        y = jax.lax.dot_general(xs, w4[e], (((1,), (1,)), ((), ())),
                                preferred_element_type=jnp.float32) + b4[e]
        return jnp.where(idcol4 == e, y, acc)

    acc = jnp.zeros((NPASS * TILE, OUTF), jnp.float32)
    acc = jax.lax.fori_loop(e_lo, e_hi + 1, body, acc)
    parts = [acc[p * TILE:(p + 1) * TILE, :] for p in range(NPASS)]
    parts.append(jnp.zeros((TILE, ROWW - NPASS * OUTF), jnp.float32))
    out_ref[...] = jnp.concatenate(parts, axis=1)


def _full(shape):
    zeros = tuple(0 for _ in shape)
    return pl.BlockSpec(shape, lambda g, ids, z=zeros: z)


def _grouped_mlp(ids_sorted, x_sorted, Ws, bs):
    b0r, b1r, b2r, b3r = (b.reshape(NUM_MODELS, 1, HID) for b in bs[:4])
    b4r = bs[4].reshape(NUM_MODELS, 1, OUTF)
    grid_spec = pltpu.PrefetchScalarGridSpec(
        num_scalar_prefetch=1,
        grid=(GRID,),
        in_specs=[
            pl.BlockSpec((TILE, ROWW), lambda g, ids: (g, 0)),
            _full((NUM_MODELS, HID, INF)),
            _full((NUM_MODELS, 1, HID)),
            _full((NUM_MODELS, HID, HID)),
            _full((NUM_MODELS, 1, HID)),
            _full((NUM_MODELS, HID, HID)),
            _full((NUM_MODELS, 1, HID)),
            _full((NUM_MODELS, HID, HID)),
            _full((NUM_MODELS, 1, HID)),
            _full((NUM_MODELS, OUTF, HID)),
            _full((NUM_MODELS, 1, OUTF)),
        ],
        out_specs=pl.BlockSpec((TILE, ROWW), lambda g, ids: (g, 0)),
    )
    return pl.pallas_call(
        _mlp_kernel,
        grid_spec=grid_spec,
        out_shape=jax.ShapeDtypeStruct((BATCH, ROWW), jnp.float32),
    )(ids_sorted, x_sorted, Ws[0], b0r, Ws[1], b1r, Ws[2], b2r, Ws[3],
      b3r, Ws[4], b4r)


def kernel(inputs, W0, b0, W1, b1, W2, b2, W3, b3, W4, b4):
    idx, logits, probs = _selection(inputs)
    perm = jnp.argsort(idx).astype(jnp.int32)
    inv = jnp.argsort(perm).astype(jnp.int32)
    xpad = jnp.concatenate(
        [inputs, idx[:, None].astype(jnp.float32),
         jnp.zeros((BATCH, ROWW - INF - 1), jnp.float32)], axis=1)
    x_sorted = _sc_row_gather(xpad, perm.reshape(-1, SC_CHUNK))
    ids_sorted = x_sorted[:, INF].astype(jnp.int32)
    y_flat = _grouped_mlp(ids_sorted, x_sorted,
                          (W0, W1, W2, W3, W4), (b0, b1, b2, b3, b4))
    out_flat = _sc_row_gather(y_flat, inv.reshape(-1, SC_CHUNK))
    model_outputs = out_flat[:, :NPASS * OUTF].reshape(
        BATCH, NPASS, OUTF).transpose(1, 0, 2)
    return (model_outputs, idx, logits, probs)


# speedup vs baseline: 777.1797x; 1.0117x over previous
import functools

import jax
import jax.numpy as jnp
import numpy as np
from jax.experimental import pallas as pl
from jax.experimental.pallas import tpu as pltpu
from jax.experimental.pallas import tpu_sc as plsc

NUM_MODELS = 64
BATCH = 8192
HID = 64
INF = 6
OUTF = 3
NPASS = 4
ROWW = 16
TILE = 128
GRID = BATCH // TILE
SEL_TILE = 1024
SC_WORKERS = 32
SC_CHUNK = 128


def _const_kernel(logit_ref, prob_ref):
    logit_ref[...] = jnp.ones_like(logit_ref)
    prob_ref[...] = jnp.full_like(prob_ref, 1.0 / NUM_MODELS)


def _constants():
    nblk = BATCH // SEL_TILE
    return pl.pallas_call(
        _const_kernel,
        grid=(nblk,),
        out_specs=[
            pl.BlockSpec((SEL_TILE, NUM_MODELS), lambda i: (i, 0)),
            pl.BlockSpec((SEL_TILE, NUM_MODELS), lambda i: (i, 0)),
        ],
        out_shape=[
            jax.ShapeDtypeStruct((BATCH, NUM_MODELS), jnp.float32),
            jax.ShapeDtypeStruct((BATCH, NUM_MODELS), jnp.float32),
        ],
    )()


RANK_CHUNK = 512


def _sel_rank_kernel(x_ref, idx_ref, xpad_ref, inv_ref):
    x = x_ref[...]
    x0 = x[:, 0:1]
    x2 = x[:, 2:3]
    ang = jnp.arctan2(x2, x0)
    ang = jnp.mod(ang + 2 * np.pi, 2 * np.pi) / (2 * np.pi) * NUM_MODELS
    idxf = jnp.floor(ang)
    idxi = idxf.astype(jnp.int32)
    idx_ref[...] = idxi
    pad = jnp.zeros((BATCH, ROWW - INF - 1), jnp.float32)
    xpad_ref[...] = jnp.concatenate([x, idxf, pad], axis=1)

    lane = jax.lax.broadcasted_iota(jnp.int32, (BATCH, NUM_MODELS), 1)
    onehot = (idxi == lane).astype(jnp.float32)
    total = jnp.sum(onehot, axis=0, keepdims=True)
    ur = jax.lax.broadcasted_iota(jnp.int32, (NUM_MODELS, NUM_MODELS), 0)
    uc = jax.lax.broadcasted_iota(jnp.int32, (NUM_MODELS, NUM_MODELS), 1)
    ustrict = (ur < uc).astype(jnp.float32)
    goff = jax.lax.dot_general(total, ustrict, (((1,), (0,)), ((), ())),
                               preferred_element_type=jnp.float32)
    tr = jax.lax.broadcasted_iota(jnp.int32, (RANK_CHUNK, RANK_CHUNK), 0)
    tc = jax.lax.broadcasted_iota(jnp.int32, (RANK_CHUNK, RANK_CHUNK), 1)
    tstrict = (tc < tr).astype(jnp.float32)
    carry = jnp.zeros((1, NUM_MODELS), jnp.float32)
    for i in range(BATCH // RANK_CHUNK):
        oc = onehot[i * RANK_CHUNK:(i + 1) * RANK_CHUNK]
        wt = jax.lax.dot_general(tstrict, oc, (((1,), (0,)), ((), ())),
                                 preferred_element_type=jnp.float32)
        pos = jnp.sum(oc * (wt + carry + goff), axis=1, keepdims=True)
        inv_ref[i * RANK_CHUNK:(i + 1) * RANK_CHUNK, :] = pos.astype(jnp.int32)
        carry = carry + jnp.sum(oc, axis=0, keepdims=True)


def _selection(inputs):
    idx, xpad, inv = pl.pallas_call(
        _sel_rank_kernel,
        in_specs=[pl.BlockSpec((BATCH, INF), lambda: (0, 0))],
        out_specs=[
            pl.BlockSpec((BATCH, 1), lambda: (0, 0)),
            pl.BlockSpec((BATCH, ROWW), lambda: (0, 0)),
            pl.BlockSpec((BATCH, 1), lambda: (0, 0)),
        ],
        out_shape=[
            jax.ShapeDtypeStruct((BATCH, 1), jnp.int32),
            jax.ShapeDtypeStruct((BATCH, ROWW), jnp.float32),
            jax.ShapeDtypeStruct((BATCH, 1), jnp.int32),
        ],
    )(inputs)
    return idx[:, 0], xpad, inv[:, 0]


def _sc_row_gather(table, idx2d):
    nrows, ncols = table.shape
    b_per_w = nrows // SC_WORKERS
    nchunk = b_per_w // SC_CHUNK
    mesh = plsc.VectorSubcoreMesh(core_axis_name="c", subcore_axis_name="s")

    @functools.partial(
        pl.kernel, mesh=mesh,
        out_type=jax.ShapeDtypeStruct((nrows, ncols), jnp.float32),
        compiler_params=pltpu.CompilerParams(use_tc_tiling_on_sc=False),
        scratch_types=[
            pltpu.VMEM((nchunk, SC_CHUNK), jnp.int32),
            pltpu.VMEM((b_per_w, ncols), jnp.float32),
            pltpu.SemaphoreType.DMA,
        ],
    )
    def k(table_hbm, idx_hbm, out_hbm, idx_v, rows_v, sem):
        wid = jax.lax.axis_index("s") * 2 + jax.lax.axis_index("c")
        base = wid * b_per_w
        pltpu.sync_copy(idx_hbm.at[pl.ds(wid * nchunk, nchunk)], idx_v)
        copies = [
            pltpu.async_copy(table_hbm.at[idx_v.at[j]],
                             rows_v.at[pl.ds(j * SC_CHUNK, SC_CHUNK)], sem)
            for j in range(nchunk)
        ]
        for c in copies:
            c.wait()
        pltpu.sync_copy(rows_v, out_hbm.at[pl.ds(base, b_per_w)])

    return k(table, idx2d)


def _sc_row_scatter(rows, idx2d):
    nrows, ncols = rows.shape
    b_per_w = nrows // SC_WORKERS
    nchunk = b_per_w // SC_CHUNK
    mesh = plsc.VectorSubcoreMesh(core_axis_name="c", subcore_axis_name="s")

    @functools.partial(
        pl.kernel, mesh=mesh,
        out_type=jax.ShapeDtypeStruct((nrows, ncols), jnp.float32),
        compiler_params=pltpu.CompilerParams(use_tc_tiling_on_sc=False),
        scratch_types=[
            pltpu.VMEM((nchunk, SC_CHUNK), jnp.int32),
            pltpu.VMEM((b_per_w, ncols), jnp.float32),
            pltpu.SemaphoreType.DMA,
        ],
    )
    def k(rows_hbm, idx_hbm, out_hbm, idx_v, rows_v, sem):
        wid = jax.lax.axis_index("s") * 2 + jax.lax.axis_index("c")
        base = wid * b_per_w
        pltpu.sync_copy(idx_hbm.at[pl.ds(wid * nchunk, nchunk)], idx_v)
        pltpu.sync_copy(rows_hbm.at[pl.ds(base, b_per_w)], rows_v)
        copies = [
            pltpu.async_copy(rows_v.at[pl.ds(j * SC_CHUNK, SC_CHUNK)],
                             out_hbm.at[idx_v.at[j]], sem)
            for j in range(nchunk)
        ]
        for c in copies:
            c.wait()

    return k(rows, idx2d)


def _mlp_kernel(ids_sm, x_ref, w0, b0, w1, b1, w2, b2, w3, b3, w4, b4,
                out_ref):
    g = pl.program_id(0)
    e_lo = ids_sm[g * TILE]
    e_hi = ids_sm[g * TILE + TILE - 1]
    x = x_ref[:, :INF]
    idcol = x_ref[:, INF:INF + 1].astype(jnp.int32)
    idcol4 = jnp.concatenate([idcol] * NPASS, axis=0)

    row = jax.lax.broadcasted_iota(jnp.int32, (NPASS * TILE, HID), 0)
    col = jax.lax.broadcasted_iota(jnp.int32, (NPASS * TILE, HID), 1)
    colmask = (col < (HID // NPASS) * (row // TILE + 1)).astype(jnp.float32)

    def body(e, acc):
        y0 = jax.lax.dot_general(x, w0[e], (((1,), (1,)), ((), ())),
                                 preferred_element_type=jnp.float32)
        y0 = jnp.maximum(y0 + b0[e], 0.0)
        xs = jnp.concatenate([y0] * NPASS, axis=0) * colmask
        for w, b in ((w1, b1), (w2, b2), (w3, b3)):
            xs = jax.lax.dot_general(xs, w[e], (((1,), (1,)), ((), ())),
                                     preferred_element_type=jnp.float32)
            xs = jnp.maximum(xs + b[e], 0.0) * colmask
        y = jax.lax.dot_general(xs, w4[e], (((1,), (1,)), ((), ())),
                                preferred_element_type=jnp.float32) + b4[e]
        return jnp.where(idcol4 == e, y, acc)

    acc = jnp.zeros((NPASS * TILE, OUTF), jnp.float32)
    acc = jax.lax.fori_loop(e_lo, e_hi + 1, body, acc)
    parts = [acc[p * TILE:(p + 1) * TILE, :] for p in range(NPASS)]
    parts.append(jnp.zeros((TILE, ROWW - NPASS * OUTF), jnp.float32))
    out_ref[...] = jnp.concatenate(parts, axis=1)


def _full(shape):
    zeros = tuple(0 for _ in shape)
    return pl.BlockSpec(shape, lambda g, ids, z=zeros: z)


def _grouped_mlp(ids_sorted, x_sorted, Ws, bs):
    b0r, b1r, b2r, b3r = (b.reshape(NUM_MODELS, 1, HID) for b in bs[:4])
    b4r = bs[4].reshape(NUM_MODELS, 1, OUTF)
    grid_spec = pltpu.PrefetchScalarGridSpec(
        num_scalar_prefetch=1,
        grid=(GRID,),
        in_specs=[
            pl.BlockSpec((TILE, ROWW), lambda g, ids: (g, 0)),
            _full((NUM_MODELS, HID, INF)),
            _full((NUM_MODELS, 1, HID)),
            _full((NUM_MODELS, HID, HID)),
            _full((NUM_MODELS, 1, HID)),
            _full((NUM_MODELS, HID, HID)),
            _full((NUM_MODELS, 1, HID)),
            _full((NUM_MODELS, HID, HID)),
            _full((NUM_MODELS, 1, HID)),
            _full((NUM_MODELS, OUTF, HID)),
            _full((NUM_MODELS, 1, OUTF)),
        ],
        out_specs=pl.BlockSpec((TILE, ROWW), lambda g, ids: (g, 0)),
    )
    return pl.pallas_call(
        _mlp_kernel,
        grid_spec=grid_spec,
        out_shape=jax.ShapeDtypeStruct((BATCH, ROWW), jnp.float32),
    )(ids_sorted, x_sorted, Ws[0], b0r, Ws[1], b1r, Ws[2], b2r, Ws[3],
      b3r, Ws[4], b4r)


def kernel(inputs, W0, b0, W1, b1, W2, b2, W3, b3, W4, b4):
    logits, probs = _constants()
    idx, xpad, inv = _selection(inputs)
    inv2d = inv.reshape(-1, SC_CHUNK)
    x_sorted = _sc_row_scatter(xpad, inv2d)
    ids_sorted = x_sorted[:, INF].astype(jnp.int32)
    y_flat = _grouped_mlp(ids_sorted, x_sorted,
                          (W0, W1, W2, W3, W4), (b0, b1, b2, b3, b4))
    out_flat = _sc_row_gather(y_flat, inv2d)
    model_outputs = out_flat[:, :NPASS * OUTF].reshape(
        BATCH, NPASS, OUTF).transpose(1, 0, 2)
    return (model_outputs, idx, logits, probs)


# speedup vs baseline: 888.8267x; 1.1437x over previous
import functools

import jax
import jax.numpy as jnp
import numpy as np
from jax.experimental import pallas as pl
from jax.experimental.pallas import tpu as pltpu
from jax.experimental.pallas import tpu_sc as plsc

NUM_MODELS = 64
BATCH = 8192
HID = 64
INF = 6
OUTF = 3
NPASS = 4
ROWW = 16
TILE = 128
GRID = BATCH // TILE
SEL_TILE = 1024
SC_WORKERS = 32
SC_CHUNK = 128


def _const_kernel(logit_ref, prob_ref):
    logit_ref[...] = jnp.ones_like(logit_ref)
    prob_ref[...] = jnp.full_like(prob_ref, 1.0 / NUM_MODELS)


def _constants():
    nblk = BATCH // SEL_TILE
    return pl.pallas_call(
        _const_kernel,
        grid=(nblk,),
        out_specs=[
            pl.BlockSpec((SEL_TILE, NUM_MODELS), lambda i: (i, 0)),
            pl.BlockSpec((SEL_TILE, NUM_MODELS), lambda i: (i, 0)),
        ],
        out_shape=[
            jax.ShapeDtypeStruct((BATCH, NUM_MODELS), jnp.float32),
            jax.ShapeDtypeStruct((BATCH, NUM_MODELS), jnp.float32),
        ],
    )()


RANK_CHUNK = 512


def _sel_rank_kernel(x_ref, idx_ref, xpad_ref, inv_ref):
    x = x_ref[...]
    x0 = x[:, 0:1]
    x2 = x[:, 2:3]
    ang = jnp.arctan2(x2, x0)
    ang = jnp.mod(ang + 2 * np.pi, 2 * np.pi) / (2 * np.pi) * NUM_MODELS
    idxf = jnp.floor(ang)
    idxi = idxf.astype(jnp.int32)
    idx_ref[...] = idxi
    pad = jnp.zeros((BATCH, ROWW - INF - 1), jnp.float32)
    xpad_ref[...] = jnp.concatenate([x, idxf, pad], axis=1)

    lane = jax.lax.broadcasted_iota(jnp.int32, (BATCH, NUM_MODELS), 1)
    onehot = (idxi == lane).astype(jnp.float32)
    total = jnp.sum(onehot, axis=0, keepdims=True)
    ur = jax.lax.broadcasted_iota(jnp.int32, (NUM_MODELS, NUM_MODELS), 0)
    uc = jax.lax.broadcasted_iota(jnp.int32, (NUM_MODELS, NUM_MODELS), 1)
    ustrict = (ur < uc).astype(jnp.float32)
    goff = jax.lax.dot_general(total, ustrict, (((1,), (0,)), ((), ())),
                               preferred_element_type=jnp.float32)
    tr = jax.lax.broadcasted_iota(jnp.int32, (RANK_CHUNK, RANK_CHUNK), 0)
    tc = jax.lax.broadcasted_iota(jnp.int32, (RANK_CHUNK, RANK_CHUNK), 1)
    tstrict = (tc < tr).astype(jnp.float32)
    carry = jnp.zeros((1, NUM_MODELS), jnp.float32)
    for i in range(BATCH // RANK_CHUNK):
        oc = onehot[i * RANK_CHUNK:(i + 1) * RANK_CHUNK]
        wt = jax.lax.dot_general(tstrict, oc, (((1,), (0,)), ((), ())),
                                 preferred_element_type=jnp.float32)
        pos = jnp.sum(oc * (wt + carry + goff), axis=1, keepdims=True)
        inv_ref[i * RANK_CHUNK:(i + 1) * RANK_CHUNK, :] = pos.astype(jnp.int32)
        carry = carry + jnp.sum(oc, axis=0, keepdims=True)


def _selection(inputs):
    idx, xpad, inv = pl.pallas_call(
        _sel_rank_kernel,
        in_specs=[pl.BlockSpec((BATCH, INF), lambda: (0, 0))],
        out_specs=[
            pl.BlockSpec((BATCH, 1), lambda: (0, 0)),
            pl.BlockSpec((BATCH, ROWW), lambda: (0, 0)),
            pl.BlockSpec((BATCH, 1), lambda: (0, 0)),
        ],
        out_shape=[
            jax.ShapeDtypeStruct((BATCH, 1), jnp.int32),
            jax.ShapeDtypeStruct((BATCH, ROWW), jnp.float32),
            jax.ShapeDtypeStruct((BATCH, 1), jnp.int32),
        ],
    )(inputs)
    return idx[:, 0], xpad, inv[:, 0]


def _sc_row_gather(table, idx2d):
    nrows, ncols = table.shape
    b_per_w = nrows // SC_WORKERS
    nchunk = b_per_w // SC_CHUNK
    mesh = plsc.VectorSubcoreMesh(core_axis_name="c", subcore_axis_name="s")

    @functools.partial(
        pl.kernel, mesh=mesh,
        out_type=jax.ShapeDtypeStruct((nrows, ncols), jnp.float32),
        compiler_params=pltpu.CompilerParams(use_tc_tiling_on_sc=False),
        scratch_types=[
            pltpu.VMEM((nchunk, SC_CHUNK), jnp.int32),
            pltpu.VMEM((b_per_w, ncols), jnp.float32),
            pltpu.SemaphoreType.DMA,
        ],
    )
    def k(table_hbm, idx_hbm, out_hbm, idx_v, rows_v, sem):
        wid = jax.lax.axis_index("s") * 2 + jax.lax.axis_index("c")
        base = wid * b_per_w
        pltpu.sync_copy(idx_hbm.at[pl.ds(wid * nchunk, nchunk)], idx_v)
        copies = [
            pltpu.async_copy(table_hbm.at[idx_v.at[j]],
                             rows_v.at[pl.ds(j * SC_CHUNK, SC_CHUNK)], sem)
            for j in range(nchunk)
        ]
        for c in copies:
            c.wait()
        pltpu.sync_copy(rows_v, out_hbm.at[pl.ds(base, b_per_w)])

    return k(table, idx2d)


def _sc_row_scatter(rows, idx2d):
    nrows, ncols = rows.shape
    b_per_w = nrows // SC_WORKERS
    nchunk = b_per_w // SC_CHUNK
    mesh = plsc.VectorSubcoreMesh(core_axis_name="c", subcore_axis_name="s")

    @functools.partial(
        pl.kernel, mesh=mesh,
        out_type=jax.ShapeDtypeStruct((nrows, ncols), jnp.float32),
        compiler_params=pltpu.CompilerParams(use_tc_tiling_on_sc=False),
        scratch_types=[
            pltpu.VMEM((nchunk, SC_CHUNK), jnp.int32),
            pltpu.VMEM((b_per_w, ncols), jnp.float32),
            pltpu.SemaphoreType.DMA,
        ],
    )
    def k(rows_hbm, idx_hbm, out_hbm, idx_v, rows_v, sem):
        wid = jax.lax.axis_index("s") * 2 + jax.lax.axis_index("c")
        base = wid * b_per_w
        pltpu.sync_copy(idx_hbm.at[pl.ds(wid * nchunk, nchunk)], idx_v)
        pltpu.sync_copy(rows_hbm.at[pl.ds(base, b_per_w)], rows_v)
        copies = [
            pltpu.async_copy(rows_v.at[pl.ds(j * SC_CHUNK, SC_CHUNK)],
                             out_hbm.at[idx_v.at[j]], sem)
            for j in range(nchunk)
        ]
        for c in copies:
            c.wait()

    return k(rows, idx2d)


def _mlp_kernel(ids_sm, cmask_ref, x_ref, w0, b0, w1, b1, w2, b2, w3, b3,
                w4, b4, out_ref):
    g = pl.program_id(0)
    e_lo = ids_sm[g * TILE]
    e_hi = ids_sm[g * TILE + TILE - 1]
    xin = x_ref[...]
    x4 = jnp.concatenate([xin[:, :INF]] * NPASS, axis=0)
    idcol = xin[:, INF:INF + 1].astype(jnp.int32)
    idcol4 = jnp.concatenate([idcol] * NPASS, axis=0)
    colmask = cmask_ref[...]

    def chain(e):
        y0 = jax.lax.dot_general(x4, w0[e], (((1,), (1,)), ((), ())),
                                 preferred_element_type=jnp.float32)
        xs = jnp.maximum(y0 + b0[e], 0.0) * colmask
        for w, b in ((w1, b1), (w2, b2), (w3, b3)):
            xs = jax.lax.dot_general(xs, w[e], (((1,), (1,)), ((), ())),
                                     preferred_element_type=jnp.float32)
            xs = jnp.maximum(xs + b[e], 0.0) * colmask
        return jax.lax.dot_general(xs, w4[e], (((1,), (1,)), ((), ())),
                                   preferred_element_type=jnp.float32) + b4[e]

    def body(i, acc):
        ea = e_lo + 2 * i
        eb = jnp.minimum(ea + 1, e_hi)
        ya = chain(ea)
        yb = chain(eb)
        acc = jnp.where(idcol4 == ea, ya, acc)
        return jnp.where(idcol4 == eb, yb, acc)

    acc = jnp.zeros((NPASS * TILE, OUTF), jnp.float32)
    acc = jax.lax.fori_loop(0, (e_hi - e_lo + 2) // 2, body, acc)
    parts = [acc[p * TILE:(p + 1) * TILE, :] for p in range(NPASS)]
    parts.append(jnp.zeros((TILE, ROWW - NPASS * OUTF), jnp.float32))
    out_ref[...] = jnp.concatenate(parts, axis=1)


def _full(shape):
    zeros = tuple(0 for _ in shape)
    return pl.BlockSpec(shape, lambda g, ids, z=zeros: z)


def _grouped_mlp(ids_sorted, x_sorted, Ws, bs):
    b0r, b1r, b2r, b3r = (b.reshape(NUM_MODELS, 1, HID) for b in bs[:4])
    b4r = bs[4].reshape(NUM_MODELS, 1, OUTF)
    rowi = np.arange(NPASS * TILE)[:, None] // TILE
    coli = np.arange(HID)[None, :]
    cmask = jnp.asarray(
        (coli < (HID // NPASS) * (rowi + 1)).astype(np.float32))
    grid_spec = pltpu.PrefetchScalarGridSpec(
        num_scalar_prefetch=1,
        grid=(GRID,),
        in_specs=[
            _full((NPASS * TILE, HID)),
            pl.BlockSpec((TILE, ROWW), lambda g, ids: (g, 0)),
            _full((NUM_MODELS, HID, INF)),
            _full((NUM_MODELS, 1, HID)),
            _full((NUM_MODELS, HID, HID)),
            _full((NUM_MODELS, 1, HID)),
            _full((NUM_MODELS, HID, HID)),
            _full((NUM_MODELS, 1, HID)),
            _full((NUM_MODELS, HID, HID)),
            _full((NUM_MODELS, 1, HID)),
            _full((NUM_MODELS, OUTF, HID)),
            _full((NUM_MODELS, 1, OUTF)),
        ],
        out_specs=pl.BlockSpec((TILE, ROWW), lambda g, ids: (g, 0)),
    )
    return pl.pallas_call(
        _mlp_kernel,
        grid_spec=grid_spec,
        out_shape=jax.ShapeDtypeStruct((BATCH, ROWW), jnp.float32),
    )(ids_sorted, cmask, x_sorted, Ws[0], b0r, Ws[1], b1r, Ws[2], b2r,
      Ws[3], b3r, Ws[4], b4r)


def kernel(inputs, W0, b0, W1, b1, W2, b2, W3, b3, W4, b4):
    logits, probs = _constants()
    idx, xpad, inv = _selection(inputs)
    inv2d = inv.reshape(-1, SC_CHUNK)
    x_sorted = _sc_row_scatter(xpad, inv2d)
    ids_sorted = x_sorted[:, INF].astype(jnp.int32)
    y_flat = _grouped_mlp(ids_sorted, x_sorted,
                          (W0, W1, W2, W3, W4), (b0, b1, b2, b3, b4))
    out_flat = _sc_row_gather(y_flat, inv2d)
    model_outputs = out_flat[:, :NPASS * OUTF].reshape(
        BATCH, NPASS, OUTF).transpose(1, 0, 2)
    return (model_outputs, idx, logits, probs)


# speedup vs baseline: 890.5452x; 1.0019x over previous
import functools

import jax
import jax.numpy as jnp
import numpy as np
from jax.experimental import pallas as pl
from jax.experimental.pallas import tpu as pltpu
from jax.experimental.pallas import tpu_sc as plsc

NUM_MODELS = 64
BATCH = 8192
HID = 64
INF = 6
OUTF = 3
NPASS = 4
ROWW = 16
TILE = 128
NSUB = 2
GRID = BATCH // (TILE * NSUB)
SEL_TILE = 1024
SC_WORKERS = 32
SC_CHUNK = 128


def _const_kernel(logit_ref, prob_ref):
    logit_ref[...] = jnp.ones_like(logit_ref)
    prob_ref[...] = jnp.full_like(prob_ref, 1.0 / NUM_MODELS)


def _constants():
    nblk = BATCH // SEL_TILE
    return pl.pallas_call(
        _const_kernel,
        grid=(nblk,),
        out_specs=[
            pl.BlockSpec((SEL_TILE, NUM_MODELS), lambda i: (i, 0)),
            pl.BlockSpec((SEL_TILE, NUM_MODELS), lambda i: (i, 0)),
        ],
        out_shape=[
            jax.ShapeDtypeStruct((BATCH, NUM_MODELS), jnp.float32),
            jax.ShapeDtypeStruct((BATCH, NUM_MODELS), jnp.float32),
        ],
    )()


RANK_CHUNK = 512


def _sel_rank_kernel(x_ref, idx_ref, xpad_ref, inv_ref):
    x = x_ref[...]
    x0 = x[:, 0:1]
    x2 = x[:, 2:3]
    ang = jnp.arctan2(x2, x0)
    ang = jnp.mod(ang + 2 * np.pi, 2 * np.pi) / (2 * np.pi) * NUM_MODELS
    idxf = jnp.floor(ang)
    idxi = idxf.astype(jnp.int32)
    idx_ref[...] = idxi
    pad = jnp.zeros((BATCH, ROWW - INF - 1), jnp.float32)
    xpad_ref[...] = jnp.concatenate([x, idxf, pad], axis=1)

    lane = jax.lax.broadcasted_iota(jnp.int32, (BATCH, NUM_MODELS), 1)
    onehot = (idxi == lane).astype(jnp.float32)
    total = jnp.sum(onehot, axis=0, keepdims=True)
    ur = jax.lax.broadcasted_iota(jnp.int32, (NUM_MODELS, NUM_MODELS), 0)
    uc = jax.lax.broadcasted_iota(jnp.int32, (NUM_MODELS, NUM_MODELS), 1)
    ustrict = (ur < uc).astype(jnp.float32)
    goff = jax.lax.dot_general(total, ustrict, (((1,), (0,)), ((), ())),
                               preferred_element_type=jnp.float32)
    tr = jax.lax.broadcasted_iota(jnp.int32, (RANK_CHUNK, RANK_CHUNK), 0)
    tc = jax.lax.broadcasted_iota(jnp.int32, (RANK_CHUNK, RANK_CHUNK), 1)
    tstrict = (tc < tr).astype(jnp.float32)
    carry = jnp.zeros((1, NUM_MODELS), jnp.float32)
    for i in range(BATCH // RANK_CHUNK):
        oc = onehot[i * RANK_CHUNK:(i + 1) * RANK_CHUNK]
        wt = jax.lax.dot_general(tstrict, oc, (((1,), (0,)), ((), ())),
                                 preferred_element_type=jnp.float32)
        pos = jnp.sum(oc * (wt + carry + goff), axis=1, keepdims=True)
        inv_ref[i * RANK_CHUNK:(i + 1) * RANK_CHUNK, :] = pos.astype(jnp.int32)
        carry = carry + jnp.sum(oc, axis=0, keepdims=True)


def _selection(inputs):
    idx, xpad, inv = pl.pallas_call(
        _sel_rank_kernel,
        in_specs=[pl.BlockSpec((BATCH, INF), lambda: (0, 0))],
        out_specs=[
            pl.BlockSpec((BATCH, 1), lambda: (0, 0)),
            pl.BlockSpec((BATCH, ROWW), lambda: (0, 0)),
            pl.BlockSpec((BATCH, 1), lambda: (0, 0)),
        ],
        out_shape=[
            jax.ShapeDtypeStruct((BATCH, 1), jnp.int32),
            jax.ShapeDtypeStruct((BATCH, ROWW), jnp.float32),
            jax.ShapeDtypeStruct((BATCH, 1), jnp.int32),
        ],
    )(inputs)
    return idx[:, 0], xpad, inv[:, 0]


def _sc_row_gather(table, idx2d):
    nrows, ncols = table.shape
    b_per_w = nrows // SC_WORKERS
    nchunk = b_per_w // SC_CHUNK
    mesh = plsc.VectorSubcoreMesh(core_axis_name="c", subcore_axis_name="s")

    @functools.partial(
        pl.kernel, mesh=mesh,
        out_type=jax.ShapeDtypeStruct((nrows, ncols), jnp.float32),
        compiler_params=pltpu.CompilerParams(use_tc_tiling_on_sc=False),
        scratch_types=[
            pltpu.VMEM((nchunk, SC_CHUNK), jnp.int32),
            pltpu.VMEM((b_per_w, ncols), jnp.float32),
            pltpu.SemaphoreType.DMA,
        ],
    )
    def k(table_hbm, idx_hbm, out_hbm, idx_v, rows_v, sem):
        wid = jax.lax.axis_index("s") * 2 + jax.lax.axis_index("c")
        base = wid * b_per_w
        pltpu.sync_copy(idx_hbm.at[pl.ds(wid * nchunk, nchunk)], idx_v)
        copies = [
            pltpu.async_copy(table_hbm.at[idx_v.at[j]],
                             rows_v.at[pl.ds(j * SC_CHUNK, SC_CHUNK)], sem)
            for j in range(nchunk)
        ]
        for c in copies:
            c.wait()
        pltpu.sync_copy(rows_v, out_hbm.at[pl.ds(base, b_per_w)])

    return k(table, idx2d)


def _sc_row_scatter(rows, idx2d):
    nrows, ncols = rows.shape
    b_per_w = nrows // SC_WORKERS
    nchunk = b_per_w // SC_CHUNK
    mesh = plsc.VectorSubcoreMesh(core_axis_name="c", subcore_axis_name="s")

    @functools.partial(
        pl.kernel, mesh=mesh,
        out_type=jax.ShapeDtypeStruct((nrows, ncols), jnp.float32),
        compiler_params=pltpu.CompilerParams(use_tc_tiling_on_sc=False),
        scratch_types=[
            pltpu.VMEM((nchunk, SC_CHUNK), jnp.int32),
            pltpu.VMEM((b_per_w, ncols), jnp.float32),
            pltpu.SemaphoreType.DMA,
        ],
    )
    def k(rows_hbm, idx_hbm, out_hbm, idx_v, rows_v, sem):
        wid = jax.lax.axis_index("s") * 2 + jax.lax.axis_index("c")
        base = wid * b_per_w
        pltpu.sync_copy(idx_hbm.at[pl.ds(wid * nchunk, nchunk)], idx_v)
        pltpu.sync_copy(rows_hbm.at[pl.ds(base, b_per_w)], rows_v)
        copies = [
            pltpu.async_copy(rows_v.at[pl.ds(j * SC_CHUNK, SC_CHUNK)],
                             out_hbm.at[idx_v.at[j]], sem)
            for j in range(nchunk)
        ]
        for c in copies:
            c.wait()

    return k(rows, idx2d)


def _mlp_kernel(ids_sm, cmask_ref, x_ref, w0, b0, w1, b1, w2, b2, w3, b3,
                w4, b4, out_ref):
    g = pl.program_id(0)
    colmask = cmask_ref[...]
    xin = x_ref[...]

    def chain(x4, e):
        y0 = jax.lax.dot_general(x4, w0[e], (((1,), (1,)), ((), ())),
                                 preferred_element_type=jnp.float32)
        xs = jnp.maximum(y0 + b0[e], 0.0) * colmask
        for w, b in ((w1, b1), (w2, b2), (w3, b3)):
            xs = jax.lax.dot_general(xs, w[e], (((1,), (1,)), ((), ())),
                                     preferred_element_type=jnp.float32)
            xs = jnp.maximum(xs + b[e], 0.0) * colmask
        return jax.lax.dot_general(xs, w4[e], (((1,), (1,)), ((), ())),
                                   preferred_element_type=jnp.float32) + b4[e]

    x4s, id4s, los, his, spans = [], [], [], [], []
    for s in range(NSUB):
        base = (g * NSUB + s) * TILE
        lo = ids_sm[base]
        hi = ids_sm[base + TILE - 1]
        xs = xin[s * TILE:(s + 1) * TILE]
        x4s.append(jnp.concatenate([xs[:, :INF]] * NPASS, axis=0))
        idc = xs[:, INF:INF + 1].astype(jnp.int32)
        id4s.append(jnp.concatenate([idc] * NPASS, axis=0))
        los.append(lo)
        his.append(hi)
        spans.append((hi - lo + 2) // 2)
    nmax = spans[0]
    for s in range(1, NSUB):
        nmax = jnp.maximum(nmax, spans[s])

    def body(i, accs):
        out = []
        for s in range(NSUB):
            ea = jnp.minimum(los[s] + 2 * i, his[s])
            eb = jnp.minimum(ea + 1, his[s])
            ya = chain(x4s[s], ea)
            yb = chain(x4s[s], eb)
            acc = jnp.where(id4s[s] == ea, ya, accs[s])
            out.append(jnp.where(id4s[s] == eb, yb, acc))
        return tuple(out)

    accs = tuple(jnp.zeros((NPASS * TILE, OUTF), jnp.float32)
                 for _ in range(NSUB))
    accs = jax.lax.fori_loop(0, nmax, body, accs)
    rows = []
    for s in range(NSUB):
        parts = [accs[s][p * TILE:(p + 1) * TILE, :] for p in range(NPASS)]
        parts.append(jnp.zeros((TILE, ROWW - NPASS * OUTF), jnp.float32))
        rows.append(jnp.concatenate(parts, axis=1))
    out_ref[...] = jnp.concatenate(rows, axis=0)


def _full(shape):
    zeros = tuple(0 for _ in shape)
    return pl.BlockSpec(shape, lambda g, ids, z=zeros: z)


def _grouped_mlp(ids_sorted, x_sorted, Ws, bs):
    b0r, b1r, b2r, b3r = (b.reshape(NUM_MODELS, 1, HID) for b in bs[:4])
    b4r = bs[4].reshape(NUM_MODELS, 1, OUTF)
    rowi = np.arange(NPASS * TILE)[:, None] // TILE
    coli = np.arange(HID)[None, :]
    cmask = jnp.asarray(
        (coli < (HID // NPASS) * (rowi + 1)).astype(np.float32))
    grid_spec = pltpu.PrefetchScalarGridSpec(
        num_scalar_prefetch=1,
        grid=(GRID,),
        in_specs=[
            _full((NPASS * TILE, HID)),
            pl.BlockSpec((NSUB * TILE, ROWW), lambda g, ids: (g, 0)),
            _full((NUM_MODELS, HID, INF)),
            _full((NUM_MODELS, 1, HID)),
            _full((NUM_MODELS, HID, HID)),
            _full((NUM_MODELS, 1, HID)),
            _full((NUM_MODELS, HID, HID)),
            _full((NUM_MODELS, 1, HID)),
            _full((NUM_MODELS, HID, HID)),
            _full((NUM_MODELS, 1, HID)),
            _full((NUM_MODELS, OUTF, HID)),
            _full((NUM_MODELS, 1, OUTF)),
        ],
        out_specs=pl.BlockSpec((NSUB * TILE, ROWW), lambda g, ids: (g, 0)),
    )
    return pl.pallas_call(
        _mlp_kernel,
        grid_spec=grid_spec,
        out_shape=jax.ShapeDtypeStruct((BATCH, ROWW), jnp.float32),
    )(ids_sorted, cmask, x_sorted, Ws[0], b0r, Ws[1], b1r, Ws[2], b2r,
      Ws[3], b3r, Ws[4], b4r)


def kernel(inputs, W0, b0, W1, b1, W2, b2, W3, b3, W4, b4):
    logits, probs = _constants()
    idx, xpad, inv = _selection(inputs)
    inv2d = inv.reshape(-1, SC_CHUNK)
    x_sorted = _sc_row_scatter(xpad, inv2d)
    ids_sorted = x_sorted[:, INF].astype(jnp.int32)
    y_flat = _grouped_mlp(ids_sorted, x_sorted,
                          (W0, W1, W2, W3, W4), (b0, b1, b2, b3, b4))
    out_flat = _sc_row_gather(y_flat, inv2d)
    model_outputs = out_flat[:, :NPASS * OUTF].reshape(
        BATCH, NPASS, OUTF).transpose(1, 0, 2)
    return (model_outputs, idx, logits, probs)


# speedup vs baseline: 984.9749x; 1.1060x over previous
import functools

import jax
import jax.numpy as jnp
import numpy as np
from jax.experimental import pallas as pl
from jax.experimental.pallas import tpu as pltpu
from jax.experimental.pallas import tpu_sc as plsc

NUM_MODELS = 64
BATCH = 8192
HID = 64
INF = 6
OUTF = 3
NPASS = 4
ROWW = 16
TILE = 128
NSUB = 2
GRID = BATCH // (TILE * NSUB)
SEL_TILE = 1024
SC_WORKERS = 32
SC_CHUNK = 128


def _const_kernel(logit_ref, prob_ref):
    logit_ref[...] = jnp.ones_like(logit_ref)
    prob_ref[...] = jnp.full_like(prob_ref, 1.0 / NUM_MODELS)


def _constants():
    nblk = BATCH // SEL_TILE
    return pl.pallas_call(
        _const_kernel,
        grid=(nblk,),
        out_specs=[
            pl.BlockSpec((SEL_TILE, NUM_MODELS), lambda i: (i, 0)),
            pl.BlockSpec((SEL_TILE, NUM_MODELS), lambda i: (i, 0)),
        ],
        out_shape=[
            jax.ShapeDtypeStruct((BATCH, NUM_MODELS), jnp.float32),
            jax.ShapeDtypeStruct((BATCH, NUM_MODELS), jnp.float32),
        ],
    )()


RANK_CHUNK = 512


def _sel_rank_kernel(xt_ref, idx_ref, inv_ref, ids_ref):
    xt = xt_ref[...]
    x0 = xt[0:1, :]
    x2 = xt[2:3, :]
    ang = jnp.arctan2(x2, x0)
    ang = jnp.mod(ang + 2 * np.pi, 2 * np.pi) / (2 * np.pi) * NUM_MODELS
    idxi = jnp.floor(ang).astype(jnp.int32)
    idx_ref[...] = idxi

    erow = jax.lax.broadcasted_iota(jnp.int32, (NUM_MODELS, BATCH), 0)
    onehot = (idxi == erow).astype(jnp.float32)
    onehot_b = onehot.astype(jnp.bfloat16)
    total = jnp.sum(onehot, axis=1, keepdims=True)
    lr = jax.lax.broadcasted_iota(jnp.int32, (NUM_MODELS, NUM_MODELS), 0)
    lc = jax.lax.broadcasted_iota(jnp.int32, (NUM_MODELS, NUM_MODELS), 1)
    lstrict = (lc < lr).astype(jnp.float32)
    lincl = (lc <= lr).astype(jnp.float32)
    goff = jax.lax.dot_general(lstrict, total, (((1,), (0,)), ((), ())),
                               preferred_element_type=jnp.float32)
    cum = jax.lax.dot_general(lincl, total, (((1,), (0,)), ((), ())),
                              preferred_element_type=jnp.float32)

    posi = jax.lax.broadcasted_iota(jnp.int32, (NUM_MODELS, BATCH), 1)
    ids_ref[...] = jnp.sum(
        (cum.astype(jnp.int32) <= posi).astype(jnp.int32),
        axis=0, keepdims=True)

    tr = jax.lax.broadcasted_iota(jnp.int32, (RANK_CHUNK, RANK_CHUNK), 0)
    tc = jax.lax.broadcasted_iota(jnp.int32, (RANK_CHUNK, RANK_CHUNK), 1)
    tupper = (tr < tc).astype(jnp.bfloat16)
    carry = jnp.zeros((NUM_MODELS, 1), jnp.float32)
    for i in range(BATCH // RANK_CHUNK):
        oc = onehot[:, i * RANK_CHUNK:(i + 1) * RANK_CHUNK]
        ocb = onehot_b[:, i * RANK_CHUNK:(i + 1) * RANK_CHUNK]
        wt = jax.lax.dot_general(ocb, tupper, (((1,), (0,)), ((), ())),
                                 preferred_element_type=jnp.float32)
        pos = jnp.sum(oc * (wt + carry + goff), axis=0, keepdims=True)
        inv_ref[0:1, i * RANK_CHUNK:(i + 1) * RANK_CHUNK] = (
            pos.astype(jnp.int32))
        carry = carry + jnp.sum(oc, axis=1, keepdims=True)


def _selection(inputs):
    xt = inputs.T
    idx, inv, ids_sorted = pl.pallas_call(
        _sel_rank_kernel,
        in_specs=[pl.BlockSpec((INF, BATCH), lambda: (0, 0))],
        out_specs=[
            pl.BlockSpec((1, BATCH), lambda: (0, 0)),
            pl.BlockSpec((1, BATCH), lambda: (0, 0)),
            pl.BlockSpec((1, BATCH), lambda: (0, 0)),
        ],
        out_shape=[
            jax.ShapeDtypeStruct((1, BATCH), jnp.int32),
            jax.ShapeDtypeStruct((1, BATCH), jnp.int32),
            jax.ShapeDtypeStruct((1, BATCH), jnp.int32),
        ],
    )(xt)
    return idx.reshape(BATCH), inv, ids_sorted.reshape(BATCH)


def _sc_row_gather(table, idx2d):
    nrows, ncols = table.shape
    b_per_w = nrows // SC_WORKERS
    nchunk = b_per_w // SC_CHUNK
    mesh = plsc.VectorSubcoreMesh(core_axis_name="c", subcore_axis_name="s")

    @functools.partial(
        pl.kernel, mesh=mesh,
        out_type=jax.ShapeDtypeStruct((nrows, ncols), jnp.float32),
        compiler_params=pltpu.CompilerParams(use_tc_tiling_on_sc=False),
        scratch_types=[
            pltpu.VMEM((nchunk, SC_CHUNK), jnp.int32),
            pltpu.VMEM((b_per_w, ncols), jnp.float32),
            pltpu.SemaphoreType.DMA,
        ],
    )
    def k(table_hbm, idx_hbm, out_hbm, idx_v, rows_v, sem):
        wid = jax.lax.axis_index("s") * 2 + jax.lax.axis_index("c")
        base = wid * b_per_w
        pltpu.sync_copy(idx_hbm.at[pl.ds(wid * nchunk, nchunk)], idx_v)
        copies = [
            pltpu.async_copy(table_hbm.at[idx_v.at[j]],
                             rows_v.at[pl.ds(j * SC_CHUNK, SC_CHUNK)], sem)
            for j in range(nchunk)
        ]
        for c in copies:
            c.wait()
        pltpu.sync_copy(rows_v, out_hbm.at[pl.ds(base, b_per_w)])

    return k(table, idx2d)


def _sc_row_scatter(rows, idx2d):
    nrows, ncols = rows.shape
    b_per_w = nrows // SC_WORKERS
    nchunk = b_per_w // SC_CHUNK
    mesh = plsc.VectorSubcoreMesh(core_axis_name="c", subcore_axis_name="s")

    @functools.partial(
        pl.kernel, mesh=mesh,
        out_type=jax.ShapeDtypeStruct((nrows, ncols), jnp.float32),
        compiler_params=pltpu.CompilerParams(use_tc_tiling_on_sc=False),
        scratch_types=[
            pltpu.VMEM((nchunk, SC_CHUNK), jnp.int32),
            pltpu.VMEM((b_per_w, ncols), jnp.float32),
            pltpu.SemaphoreType.DMA,
        ],
    )
    def k(rows_hbm, idx_hbm, out_hbm, idx_v, rows_v, sem):
        wid = jax.lax.axis_index("s") * 2 + jax.lax.axis_index("c")
        base = wid * b_per_w
        pltpu.sync_copy(idx_hbm.at[pl.ds(wid * nchunk, nchunk)], idx_v)
        pltpu.sync_copy(rows_hbm.at[pl.ds(base, b_per_w)], rows_v)
        copies = [
            pltpu.async_copy(rows_v.at[pl.ds(j * SC_CHUNK, SC_CHUNK)],
                             out_hbm.at[idx_v.at[j]], sem)
            for j in range(nchunk)
        ]
        for c in copies:
            c.wait()

    return k(rows, idx2d)


def _mlp_kernel(ids_sm, cmask_ref, ids3_ref, x_ref, w0, b0, w1, b1, w2, b2,
                w3, b3, w4, b4, out_ref):
    g = pl.program_id(0)
    colmask = cmask_ref[...]
    xin = x_ref[...]

    def chain(x4, e):
        y0 = jax.lax.dot_general(x4, w0[e], (((1,), (1,)), ((), ())),
                                 preferred_element_type=jnp.float32)
        xs = jnp.maximum(y0 + b0[e], 0.0) * colmask
        for w, b in ((w1, b1), (w2, b2), (w3, b3)):
            xs = jax.lax.dot_general(xs, w[e], (((1,), (1,)), ((), ())),
                                     preferred_element_type=jnp.float32)
            xs = jnp.maximum(xs + b[e], 0.0) * colmask
        return jax.lax.dot_general(xs, w4[e], (((1,), (1,)), ((), ())),
                                   preferred_element_type=jnp.float32) + b4[e]

    x4s, id4s, los, his, spans = [], [], [], [], []
    for s in range(NSUB):
        base = (g * NSUB + s) * TILE
        lo = ids_sm[base]
        hi = ids_sm[base + TILE - 1]
        xs = xin[s * TILE:(s + 1) * TILE]
        x4s.append(jnp.concatenate([xs[:, :INF]] * NPASS, axis=0))
        idrow = ids3_ref[0, 0:1, s * TILE:(s + 1) * TILE]
        idc = jnp.transpose(idrow, (1, 0))
        id4s.append(jnp.concatenate([idc] * NPASS, axis=0))
        los.append(lo)
        his.append(hi)
        spans.append((hi - lo + 2) // 2)
    nmax = spans[0]
    for s in range(1, NSUB):
        nmax = jnp.maximum(nmax, spans[s])

    def body(i, accs):
        out = []
        for s in range(NSUB):
            ea = jnp.minimum(los[s] + 2 * i, his[s])
            eb = jnp.minimum(ea + 1, his[s])
            ya = chain(x4s[s], ea)
            yb = chain(x4s[s], eb)
            acc = jnp.where(id4s[s] == ea, ya, accs[s])
            out.append(jnp.where(id4s[s] == eb, yb, acc))
        return tuple(out)

    accs = tuple(jnp.zeros((NPASS * TILE, OUTF), jnp.float32)
                 for _ in range(NSUB))
    accs = jax.lax.fori_loop(0, nmax, body, accs)
    rows = []
    for s in range(NSUB):
        parts = [accs[s][p * TILE:(p + 1) * TILE, :] for p in range(NPASS)]
        parts.append(jnp.zeros((TILE, ROWW - NPASS * OUTF), jnp.float32))
        rows.append(jnp.concatenate(parts, axis=1))
    out_ref[...] = jnp.concatenate(rows, axis=0)


def _full(shape):
    zeros = tuple(0 for _ in shape)
    return pl.BlockSpec(shape, lambda g, ids, z=zeros: z)


def _grouped_mlp(ids_sorted, ids3, x_sorted, Ws, bs):
    b0r, b1r, b2r, b3r = (b.reshape(NUM_MODELS, 1, HID) for b in bs[:4])
    b4r = bs[4].reshape(NUM_MODELS, 1, OUTF)
    rowi = np.arange(NPASS * TILE)[:, None] // TILE
    coli = np.arange(HID)[None, :]
    cmask = jnp.asarray(
        (coli < (HID // NPASS) * (rowi + 1)).astype(np.float32))
    grid_spec = pltpu.PrefetchScalarGridSpec(
        num_scalar_prefetch=1,
        grid=(GRID,),
        in_specs=[
            _full((NPASS * TILE, HID)),
            pl.BlockSpec((1, 1, NSUB * TILE), lambda g, ids: (g, 0, 0)),
            pl.BlockSpec((NSUB * TILE, ROWW), lambda g, ids: (g, 0)),
            _full((NUM_MODELS, HID, INF)),
            _full((NUM_MODELS, 1, HID)),
            _full((NUM_MODELS, HID, HID)),
            _full((NUM_MODELS, 1, HID)),
            _full((NUM_MODELS, HID, HID)),
            _full((NUM_MODELS, 1, HID)),
            _full((NUM_MODELS, HID, HID)),
            _full((NUM_MODELS, 1, HID)),
            _full((NUM_MODELS, OUTF, HID)),
            _full((NUM_MODELS, 1, OUTF)),
        ],
        out_specs=pl.BlockSpec((NSUB * TILE, ROWW), lambda g, ids: (g, 0)),
    )
    return pl.pallas_call(
        _mlp_kernel,
        grid_spec=grid_spec,
        out_shape=jax.ShapeDtypeStruct((BATCH, ROWW), jnp.float32),
    )(ids_sorted, cmask, ids3, x_sorted, Ws[0], b0r, Ws[1], b1r, Ws[2],
      b2r, Ws[3], b3r, Ws[4], b4r)


def kernel(inputs, W0, b0, W1, b1, W2, b2, W3, b3, W4, b4):
    logits, probs = _constants()
    idx, inv, ids_sorted = _selection(inputs)
    inv2d = inv.reshape(-1, SC_CHUNK)
    xpad = jnp.pad(inputs, ((0, 0), (0, ROWW - INF)))
    x_sorted = _sc_row_scatter(xpad, inv2d)
    ids3 = ids_sorted.reshape(GRID, 1, NSUB * TILE)
    y_flat = _grouped_mlp(ids_sorted, ids3, x_sorted,
                          (W0, W1, W2, W3, W4), (b0, b1, b2, b3, b4))
    out_flat = _sc_row_gather(y_flat, inv2d)
    model_outputs = out_flat[:, :NPASS * OUTF].reshape(
        BATCH, NPASS, OUTF).transpose(1, 0, 2)
    return (model_outputs, idx, logits, probs)


# speedup vs baseline: 987.3630x; 1.0024x over previous
import functools

import jax
import jax.numpy as jnp
import numpy as np
from jax.experimental import pallas as pl
from jax.experimental.pallas import tpu as pltpu
from jax.experimental.pallas import tpu_sc as plsc

NUM_MODELS = 64
BATCH = 8192
HID = 64
INF = 6
OUTF = 3
NPASS = 4
ROWW = 16
TILE = 128
NSUB = 2
GRID = BATCH // (TILE * NSUB)
SEL_TILE = 1024
SC_WORKERS = 32
SC_CHUNK = 128


def _const_kernel(logit_ref, prob_ref):
    logit_ref[...] = jnp.ones_like(logit_ref)
    prob_ref[...] = jnp.full_like(prob_ref, 1.0 / NUM_MODELS)


def _constants():
    nblk = BATCH // SEL_TILE
    return pl.pallas_call(
        _const_kernel,
        grid=(nblk,),
        out_specs=[
            pl.BlockSpec((SEL_TILE, NUM_MODELS), lambda i: (i, 0)),
            pl.BlockSpec((SEL_TILE, NUM_MODELS), lambda i: (i, 0)),
        ],
        out_shape=[
            jax.ShapeDtypeStruct((BATCH, NUM_MODELS), jnp.float32),
            jax.ShapeDtypeStruct((BATCH, NUM_MODELS), jnp.float32),
        ],
    )()


RANK_CHUNK = 512


def _sel_rank_kernel(xt_ref, idx_ref, inv_ref, ids_ref):
    xt = xt_ref[...]
    x0 = xt[0:1, :]
    x2 = xt[2:3, :]
    ang = jnp.arctan2(x2, x0)
    ang = jnp.mod(ang + 2 * np.pi, 2 * np.pi) / (2 * np.pi) * NUM_MODELS
    idxi = jnp.floor(ang).astype(jnp.int32)
    idx_ref[...] = idxi

    erow = jax.lax.broadcasted_iota(jnp.int32, (NUM_MODELS, BATCH), 0)
    onehot = (idxi == erow).astype(jnp.float32)
    onehot_b = onehot.astype(jnp.bfloat16)
    total = jnp.sum(onehot, axis=1, keepdims=True)
    lr = jax.lax.broadcasted_iota(jnp.int32, (NUM_MODELS, NUM_MODELS), 0)
    lc = jax.lax.broadcasted_iota(jnp.int32, (NUM_MODELS, NUM_MODELS), 1)
    lstrict = (lc < lr).astype(jnp.float32)
    lincl = (lc <= lr).astype(jnp.float32)
    goff = jax.lax.dot_general(lstrict, total, (((1,), (0,)), ((), ())),
                               preferred_element_type=jnp.float32)
    cum = jax.lax.dot_general(lincl, total, (((1,), (0,)), ((), ())),
                              preferred_element_type=jnp.float32)

    posi = jax.lax.broadcasted_iota(jnp.int32, (NUM_MODELS, BATCH), 1)
    ids_ref[...] = jnp.sum(
        (cum.astype(jnp.int32) <= posi).astype(jnp.int32),
        axis=0, keepdims=True)

    tr = jax.lax.broadcasted_iota(jnp.int32, (RANK_CHUNK, RANK_CHUNK), 0)
    tc = jax.lax.broadcasted_iota(jnp.int32, (RANK_CHUNK, RANK_CHUNK), 1)
    tupper = (tr < tc).astype(jnp.bfloat16)
    carry = jnp.zeros((NUM_MODELS, 1), jnp.float32)
    for i in range(BATCH // RANK_CHUNK):
        oc = onehot[:, i * RANK_CHUNK:(i + 1) * RANK_CHUNK]
        ocb = onehot_b[:, i * RANK_CHUNK:(i + 1) * RANK_CHUNK]
        wt = jax.lax.dot_general(ocb, tupper, (((1,), (0,)), ((), ())),
                                 preferred_element_type=jnp.float32)
        pos = jnp.sum(oc * (wt + carry + goff), axis=0, keepdims=True)
        inv_ref[0:1, i * RANK_CHUNK:(i + 1) * RANK_CHUNK] = (
            pos.astype(jnp.int32))
        carry = carry + jnp.sum(oc, axis=1, keepdims=True)


def _selection(inputs):
    xt = inputs.T
    idx, inv, ids_sorted = pl.pallas_call(
        _sel_rank_kernel,
        in_specs=[pl.BlockSpec((INF, BATCH), lambda: (0, 0))],
        out_specs=[
            pl.BlockSpec((1, BATCH), lambda: (0, 0)),
            pl.BlockSpec((1, BATCH), lambda: (0, 0)),
            pl.BlockSpec((1, BATCH), lambda: (0, 0)),
        ],
        out_shape=[
            jax.ShapeDtypeStruct((1, BATCH), jnp.int32),
            jax.ShapeDtypeStruct((1, BATCH), jnp.int32),
            jax.ShapeDtypeStruct((1, BATCH), jnp.int32),
        ],
    )(xt)
    return idx.reshape(BATCH), inv, ids_sorted.reshape(BATCH)


def _sc_row_gather(table, idx2d):
    nrows, ncols = table.shape
    b_per_w = nrows // SC_WORKERS
    nchunk = b_per_w // SC_CHUNK
    mesh = plsc.VectorSubcoreMesh(core_axis_name="c", subcore_axis_name="s")

    @functools.partial(
        pl.kernel, mesh=mesh,
        out_type=jax.ShapeDtypeStruct((nrows, ncols), jnp.float32),
        compiler_params=pltpu.CompilerParams(use_tc_tiling_on_sc=False),
        scratch_types=[
            pltpu.VMEM((nchunk, SC_CHUNK), jnp.int32),
            pltpu.VMEM((b_per_w, ncols), jnp.float32),
            pltpu.SemaphoreType.DMA,
        ],
    )
    def k(table_hbm, idx_hbm, out_hbm, idx_v, rows_v, sem):
        wid = jax.lax.axis_index("s") * 2 + jax.lax.axis_index("c")
        base = wid * b_per_w
        pltpu.sync_copy(idx_hbm.at[pl.ds(wid * nchunk, nchunk)], idx_v)
        copies = [
            pltpu.async_copy(table_hbm.at[idx_v.at[j]],
                             rows_v.at[pl.ds(j * SC_CHUNK, SC_CHUNK)], sem)
            for j in range(nchunk)
        ]
        for c in copies:
            c.wait()
        pltpu.sync_copy(rows_v, out_hbm.at[pl.ds(base, b_per_w)])

    return k(table, idx2d)


def _sc_row_scatter(rows, idx2d):
    nrows, ncols = rows.shape
    b_per_w = nrows // SC_WORKERS
    nchunk = b_per_w // SC_CHUNK
    mesh = plsc.VectorSubcoreMesh(core_axis_name="c", subcore_axis_name="s")

    @functools.partial(
        pl.kernel, mesh=mesh,
        out_type=jax.ShapeDtypeStruct((nrows, ncols), jnp.float32),
        compiler_params=pltpu.CompilerParams(use_tc_tiling_on_sc=False),
        scratch_types=[
            pltpu.VMEM((nchunk, SC_CHUNK), jnp.int32),
            pltpu.VMEM((b_per_w, ncols), jnp.float32),
            pltpu.SemaphoreType.DMA,
        ],
    )
    def k(rows_hbm, idx_hbm, out_hbm, idx_v, rows_v, sem):
        wid = jax.lax.axis_index("s") * 2 + jax.lax.axis_index("c")
        base = wid * b_per_w
        pltpu.sync_copy(idx_hbm.at[pl.ds(wid * nchunk, nchunk)], idx_v)
        pltpu.sync_copy(rows_hbm.at[pl.ds(base, b_per_w)], rows_v)
        copies = [
            pltpu.async_copy(rows_v.at[pl.ds(j * SC_CHUNK, SC_CHUNK)],
                             out_hbm.at[idx_v.at[j]], sem)
            for j in range(nchunk)
        ]
        for c in copies:
            c.wait()

    return k(rows, idx2d)


def _mlp_kernel(ids_sm, ids3_ref, x_ref, w0, b0, w1, b1, w2, b2,
                w3, b3, w4, b4, out_ref):
    g = pl.program_id(0)
    xin = x_ref[...]

    lane = jax.lax.broadcasted_iota(jnp.int32, (1, NPASS * HID), 1)
    pmask = ((lane % HID) <
             (HID // NPASS) * (lane // HID + 1)).astype(jnp.float32)

    zhh = jnp.zeros((HID, HID), jnp.float32)
    zoh = jnp.zeros((OUTF, HID), jnp.float32)

    def bdiag(w, zero):
        rows = []
        for j in range(NPASS):
            blocks = [zero] * NPASS
            blocks[j] = w
            rows.append(jnp.concatenate(blocks, axis=1))
        return jnp.concatenate(rows, axis=0)

    def chain(x6, e):
        w0c = jnp.concatenate([w0[e]] * NPASS, axis=0)
        b0c = jnp.concatenate([b0[e]] * NPASS, axis=1)
        y = jax.lax.dot_general(x6, w0c, (((1,), (1,)), ((), ())),
                                preferred_element_type=jnp.float32)
        xs = jnp.maximum(y + b0c, 0.0) * pmask
        for w, b in ((w1, b1), (w2, b2), (w3, b3)):
            wb = bdiag(w[e], zhh)
            bc = jnp.concatenate([b[e]] * NPASS, axis=1)
            y = jax.lax.dot_general(xs, wb, (((1,), (1,)), ((), ())),
                                    preferred_element_type=jnp.float32)
            xs = jnp.maximum(y + bc, 0.0) * pmask
        w4c = bdiag(w4[e], zoh)
        b4c = jnp.concatenate([b4[e]] * NPASS, axis=1)
        return jax.lax.dot_general(xs, w4c, (((1,), (1,)), ((), ())),
                                   preferred_element_type=jnp.float32) + b4c

    x6s, idcs, los, his, spans = [], [], [], [], []
    for s in range(NSUB):
        base = (g * NSUB + s) * TILE
        lo = ids_sm[base]
        hi = ids_sm[base + TILE - 1]
        xs = xin[s * TILE:(s + 1) * TILE]
        x6s.append(xs[:, :INF])
        idrow = ids3_ref[0, 0:1, s * TILE:(s + 1) * TILE]
        idcs.append(jnp.transpose(idrow, (1, 0)))
        los.append(lo)
        his.append(hi)
        spans.append((hi - lo + 2) // 2)
    nmax = spans[0]
    for s in range(1, NSUB):
        nmax = jnp.maximum(nmax, spans[s])

    def body(i, accs):
        out = []
        for s in range(NSUB):
            ea = jnp.minimum(los[s] + 2 * i, his[s])
            eb = jnp.minimum(ea + 1, his[s])
            ya = chain(x6s[s], ea)
            yb = chain(x6s[s], eb)
            acc = jnp.where(idcs[s] == ea, ya, accs[s])
            out.append(jnp.where(idcs[s] == eb, yb, acc))
        return tuple(out)

    accs = tuple(jnp.zeros((TILE, NPASS * OUTF), jnp.float32)
                 for _ in range(NSUB))
    accs = jax.lax.fori_loop(0, nmax, body, accs)
    zpad = jnp.zeros((TILE, ROWW - NPASS * OUTF), jnp.float32)
    rows = [jnp.concatenate([accs[s], zpad], axis=1) for s in range(NSUB)]
    out_ref[...] = jnp.concatenate(rows, axis=0)


def _full(shape):
    zeros = tuple(0 for _ in shape)
    return pl.BlockSpec(shape, lambda g, ids, z=zeros: z)


def _grouped_mlp(ids_sorted, ids3, x_sorted, Ws, bs):
    b0r, b1r, b2r, b3r = (b.reshape(NUM_MODELS, 1, HID) for b in bs[:4])
    b4r = bs[4].reshape(NUM_MODELS, 1, OUTF)
    grid_spec = pltpu.PrefetchScalarGridSpec(
        num_scalar_prefetch=1,
        grid=(GRID,),
        in_specs=[
            pl.BlockSpec((1, 1, NSUB * TILE), lambda g, ids: (g, 0, 0)),
            pl.BlockSpec((NSUB * TILE, ROWW), lambda g, ids: (g, 0)),
            _full((NUM_MODELS, HID, INF)),
            _full((NUM_MODELS, 1, HID)),
            _full((NUM_MODELS, HID, HID)),
            _full((NUM_MODELS, 1, HID)),
            _full((NUM_MODELS, HID, HID)),
            _full((NUM_MODELS, 1, HID)),
            _full((NUM_MODELS, HID, HID)),
            _full((NUM_MODELS, 1, HID)),
            _full((NUM_MODELS, OUTF, HID)),
            _full((NUM_MODELS, 1, OUTF)),
        ],
        out_specs=pl.BlockSpec((NSUB * TILE, ROWW), lambda g, ids: (g, 0)),
    )
    return pl.pallas_call(
        _mlp_kernel,
        grid_spec=grid_spec,
        out_shape=jax.ShapeDtypeStruct((BATCH, ROWW), jnp.float32),
    )(ids_sorted, ids3, x_sorted, Ws[0], b0r, Ws[1], b1r, Ws[2],
      b2r, Ws[3], b3r, Ws[4], b4r)


def kernel(inputs, W0, b0, W1, b1, W2, b2, W3, b3, W4, b4):
    logits, probs = _constants()
    idx, inv, ids_sorted = _selection(inputs)
    inv2d = inv.reshape(-1, SC_CHUNK)
    xpad = jnp.pad(inputs, ((0, 0), (0, ROWW - INF)))
    x_sorted = _sc_row_scatter(xpad, inv2d)
    ids3 = ids_sorted.reshape(GRID, 1, NSUB * TILE)
    y_flat = _grouped_mlp(ids_sorted, ids3, x_sorted,
                          (W0, W1, W2, W3, W4), (b0, b1, b2, b3, b4))
    out_flat = _sc_row_gather(y_flat, inv2d)
    model_outputs = out_flat[:, :NPASS * OUTF].reshape(
        BATCH, NPASS, OUTF).transpose(1, 0, 2)
    return (model_outputs, idx, logits, probs)


# speedup vs baseline: 1437.1481x; 1.4555x over previous
import functools

import jax
import jax.numpy as jnp
import numpy as np
from jax.experimental import pallas as pl
from jax.experimental.pallas import tpu as pltpu
from jax.experimental.pallas import tpu_sc as plsc

NUM_MODELS = 64
BATCH = 8192
HID = 64
INF = 6
OUTF = 3
NPASS = 4
ROWW = 16
TILE = 128
NSUB = 2
GRID = BATCH // (TILE * NSUB)
SEL_TILE = 1024
SC_WORKERS = 32
SC_CHUNK = 128


def _const_kernel(logit_ref, prob_ref):
    logit_ref[...] = jnp.ones_like(logit_ref)
    prob_ref[...] = jnp.full_like(prob_ref, 1.0 / NUM_MODELS)


def _constants():
    nblk = BATCH // SEL_TILE
    return pl.pallas_call(
        _const_kernel,
        grid=(nblk,),
        out_specs=[
            pl.BlockSpec((SEL_TILE, NUM_MODELS), lambda i: (i, 0)),
            pl.BlockSpec((SEL_TILE, NUM_MODELS), lambda i: (i, 0)),
        ],
        out_shape=[
            jax.ShapeDtypeStruct((BATCH, NUM_MODELS), jnp.float32),
            jax.ShapeDtypeStruct((BATCH, NUM_MODELS), jnp.float32),
        ],
    )()


RANK_CHUNK = 512


def _sel_rank_kernel(xt_ref, idx_ref, inv_ref, ids_ref):
    xt = xt_ref[...]
    x0 = xt[0:1, :]
    x2 = xt[2:3, :]
    ang = jnp.arctan2(x2, x0)
    ang = jnp.mod(ang + 2 * np.pi, 2 * np.pi) / (2 * np.pi) * NUM_MODELS
    idxi = jnp.floor(ang).astype(jnp.int32)
    idx_ref[...] = idxi

    erow = jax.lax.broadcasted_iota(jnp.int32, (NUM_MODELS, BATCH), 0)
    onehot = (idxi == erow).astype(jnp.float32)
    onehot_b = onehot.astype(jnp.bfloat16)
    total = jnp.sum(onehot, axis=1, keepdims=True)
    lr = jax.lax.broadcasted_iota(jnp.int32, (NUM_MODELS, NUM_MODELS), 0)
    lc = jax.lax.broadcasted_iota(jnp.int32, (NUM_MODELS, NUM_MODELS), 1)
    lstrict = (lc < lr).astype(jnp.float32)
    lincl = (lc <= lr).astype(jnp.float32)
    goff = jax.lax.dot_general(lstrict, total, (((1,), (0,)), ((), ())),
                               preferred_element_type=jnp.float32)
    cum = jax.lax.dot_general(lincl, total, (((1,), (0,)), ((), ())),
                              preferred_element_type=jnp.float32)

    posi = jax.lax.broadcasted_iota(jnp.int32, (NUM_MODELS, BATCH), 1)
    ids_ref[...] = jnp.sum(
        (cum.astype(jnp.int32) <= posi).astype(jnp.int32),
        axis=0, keepdims=True)

    tr = jax.lax.broadcasted_iota(jnp.int32, (RANK_CHUNK, RANK_CHUNK), 0)
    tc = jax.lax.broadcasted_iota(jnp.int32, (RANK_CHUNK, RANK_CHUNK), 1)
    tupper = (tr < tc).astype(jnp.bfloat16)
    carry = jnp.zeros((NUM_MODELS, 1), jnp.float32)
    for i in range(BATCH // RANK_CHUNK):
        oc = onehot[:, i * RANK_CHUNK:(i + 1) * RANK_CHUNK]
        ocb = onehot_b[:, i * RANK_CHUNK:(i + 1) * RANK_CHUNK]
        wt = jax.lax.dot_general(ocb, tupper, (((1,), (0,)), ((), ())),
                                 preferred_element_type=jnp.float32)
        pos = jnp.sum(oc * (wt + carry + goff), axis=0, keepdims=True)
        inv_ref[0:1, i * RANK_CHUNK:(i + 1) * RANK_CHUNK] = (
            pos.astype(jnp.int32))
        carry = carry + jnp.sum(oc, axis=1, keepdims=True)


def _selection(inputs):
    xt = inputs.T
    idx, inv, ids_sorted = pl.pallas_call(
        _sel_rank_kernel,
        in_specs=[pl.BlockSpec((INF, BATCH), lambda: (0, 0))],
        out_specs=[
            pl.BlockSpec((1, BATCH), lambda: (0, 0)),
            pl.BlockSpec((1, BATCH), lambda: (0, 0)),
            pl.BlockSpec((1, BATCH), lambda: (0, 0)),
        ],
        out_shape=[
            jax.ShapeDtypeStruct((1, BATCH), jnp.int32),
            jax.ShapeDtypeStruct((1, BATCH), jnp.int32),
            jax.ShapeDtypeStruct((1, BATCH), jnp.int32),
        ],
    )(xt)
    return idx.reshape(BATCH), inv, ids_sorted.reshape(BATCH)


def _sc_row_gather(table, idx2d):
    nrows, ncols = table.shape
    b_per_w = nrows // SC_WORKERS
    nchunk = b_per_w // SC_CHUNK
    mesh = plsc.VectorSubcoreMesh(core_axis_name="c", subcore_axis_name="s")

    @functools.partial(
        pl.kernel, mesh=mesh,
        out_type=jax.ShapeDtypeStruct((nrows, ncols), jnp.float32),
        compiler_params=pltpu.CompilerParams(use_tc_tiling_on_sc=False),
        scratch_types=[
            pltpu.VMEM((nchunk, SC_CHUNK), jnp.int32),
            pltpu.VMEM((b_per_w, ncols), jnp.float32),
            pltpu.SemaphoreType.DMA,
        ],
    )
    def k(table_hbm, idx_hbm, out_hbm, idx_v, rows_v, sem):
        wid = jax.lax.axis_index("s") * 2 + jax.lax.axis_index("c")
        base = wid * b_per_w
        pltpu.sync_copy(idx_hbm.at[pl.ds(wid * nchunk, nchunk)], idx_v)
        copies = [
            pltpu.async_copy(table_hbm.at[idx_v.at[j]],
                             rows_v.at[pl.ds(j * SC_CHUNK, SC_CHUNK)], sem)
            for j in range(nchunk)
        ]
        for c in copies:
            c.wait()
        pltpu.sync_copy(rows_v, out_hbm.at[pl.ds(base, b_per_w)])

    return k(table, idx2d)


def _sc_row_scatter(rows, idx2d):
    nrows, ncols = rows.shape
    b_per_w = nrows // SC_WORKERS
    nchunk = b_per_w // SC_CHUNK
    mesh = plsc.VectorSubcoreMesh(core_axis_name="c", subcore_axis_name="s")

    @functools.partial(
        pl.kernel, mesh=mesh,
        out_type=jax.ShapeDtypeStruct((nrows, ncols), jnp.float32),
        compiler_params=pltpu.CompilerParams(use_tc_tiling_on_sc=False),
        scratch_types=[
            pltpu.VMEM((nchunk, SC_CHUNK), jnp.int32),
            pltpu.VMEM((b_per_w, ncols), jnp.float32),
            pltpu.SemaphoreType.DMA,
        ],
    )
    def k(rows_hbm, idx_hbm, out_hbm, idx_v, rows_v, sem):
        wid = jax.lax.axis_index("s") * 2 + jax.lax.axis_index("c")
        base = wid * b_per_w
        pltpu.sync_copy(idx_hbm.at[pl.ds(wid * nchunk, nchunk)], idx_v)
        pltpu.sync_copy(rows_hbm.at[pl.ds(base, b_per_w)], rows_v)
        copies = [
            pltpu.async_copy(rows_v.at[pl.ds(j * SC_CHUNK, SC_CHUNK)],
                             out_hbm.at[idx_v.at[j]], sem)
            for j in range(nchunk)
        ]
        for c in copies:
            c.wait()

    return k(rows, idx2d)


def _mlp_kernel(ids_sm, ids3_ref, x_ref, w0, b0, w1, b1, w2, b2,
                w3, b3, w4, b4, out_ref):
    g = pl.program_id(0)
    xin = x_ref[...]

    lane = jax.lax.broadcasted_iota(jnp.int32, (1, NPASS * HID), 1)
    pmask = ((lane % HID) <
             (HID // NPASS) * (lane // HID + 1)).astype(jnp.float32)

    zhh = jnp.zeros((HID, HID), jnp.float32)
    zoh = jnp.zeros((OUTF, HID), jnp.float32)

    def bdiag(w, zero):
        rows = []
        for j in range(NPASS):
            blocks = [zero] * NPASS
            blocks[j] = w
            rows.append(jnp.concatenate(blocks, axis=1))
        return jnp.concatenate(rows, axis=0)

    x6s, idcs, los, his, spans = [], [], [], [], []
    for s in range(NSUB):
        base = (g * NSUB + s) * TILE
        lo = ids_sm[base]
        hi = ids_sm[base + TILE - 1]
        xs = xin[s * TILE:(s + 1) * TILE]
        x6s.append(xs[:, :INF])
        idrow = ids3_ref[0, 0:1, s * TILE:(s + 1) * TILE]
        idcs.append(jnp.transpose(idrow, (1, 0)))
        los.append(lo)
        his.append(hi)
        spans.append((hi - lo + 2) // 2)
    nmax = spans[0]
    for s in range(1, NSUB):
        nmax = jnp.maximum(nmax, spans[s])

    def body(i, accs):
        chains = []
        for s in range(NSUB):
            ea = jnp.minimum(los[s] + 2 * i, his[s])
            eb = jnp.minimum(ea + 1, his[s])
            chains.append((s, ea))
            chains.append((s, eb))
        xs_l = []
        for s, e in chains:
            w0c = jnp.concatenate([w0[e]] * NPASS, axis=0)
            b0c = jnp.concatenate([b0[e]] * NPASS, axis=1)
            y = jax.lax.dot_general(x6s[s], w0c, (((1,), (1,)), ((), ())),
                                    preferred_element_type=jnp.float32)
            xs_l.append(jnp.maximum(y + b0c, 0.0) * pmask)
        for w, b in ((w1, b1), (w2, b2), (w3, b3)):
            nxt = []
            for (s, e), xs in zip(chains, xs_l):
                wb = bdiag(w[e], zhh)
                bc = jnp.concatenate([b[e]] * NPASS, axis=1)
                y = jax.lax.dot_general(xs, wb, (((1,), (1,)), ((), ())),
                                        preferred_element_type=jnp.float32)
                nxt.append(jnp.maximum(y + bc, 0.0) * pmask)
            xs_l = nxt
        out = list(accs)
        for (s, e), xs in zip(chains, xs_l):
            w4c = bdiag(w4[e], zoh)
            b4c = jnp.concatenate([b4[e]] * NPASS, axis=1)
            y = jax.lax.dot_general(xs, w4c, (((1,), (1,)), ((), ())),
                                    preferred_element_type=jnp.float32) + b4c
            out[s] = jnp.where(idcs[s] == e, y, out[s])
        return tuple(out)

    accs = tuple(jnp.zeros((TILE, NPASS * OUTF), jnp.float32)
                 for _ in range(NSUB))
    accs = jax.lax.fori_loop(0, nmax, body, accs)
    zpad = jnp.zeros((TILE, ROWW - NPASS * OUTF), jnp.float32)
    rows = [jnp.concatenate([accs[s], zpad], axis=1) for s in range(NSUB)]
    out_ref[...] = jnp.concatenate(rows, axis=0)


def _full(shape):
    zeros = tuple(0 for _ in shape)
    return pl.BlockSpec(shape, lambda g, ids, z=zeros: z)


def _grouped_mlp(ids_sorted, ids3, x_sorted, Ws, bs):
    b0r, b1r, b2r, b3r = (b.reshape(NUM_MODELS, 1, HID) for b in bs[:4])
    b4r = bs[4].reshape(NUM_MODELS, 1, OUTF)
    grid_spec = pltpu.PrefetchScalarGridSpec(
        num_scalar_prefetch=1,
        grid=(GRID,),
        in_specs=[
            pl.BlockSpec((1, 1, NSUB * TILE), lambda g, ids: (g, 0, 0)),
            pl.BlockSpec((NSUB * TILE, ROWW), lambda g, ids: (g, 0)),
            _full((NUM_MODELS, HID, INF)),
            _full((NUM_MODELS, 1, HID)),
            _full((NUM_MODELS, HID, HID)),
            _full((NUM_MODELS, 1, HID)),
            _full((NUM_MODELS, HID, HID)),
            _full((NUM_MODELS, 1, HID)),
            _full((NUM_MODELS, HID, HID)),
            _full((NUM_MODELS, 1, HID)),
            _full((NUM_MODELS, OUTF, HID)),
            _full((NUM_MODELS, 1, OUTF)),
        ],
        out_specs=pl.BlockSpec((NSUB * TILE, ROWW), lambda g, ids: (g, 0)),
    )
    return pl.pallas_call(
        _mlp_kernel,
        grid_spec=grid_spec,
        out_shape=jax.ShapeDtypeStruct((BATCH, ROWW), jnp.float32),
    )(ids_sorted, ids3, x_sorted, Ws[0], b0r, Ws[1], b1r, Ws[2],
      b2r, Ws[3], b3r, Ws[4], b4r)


def kernel(inputs, W0, b0, W1, b1, W2, b2, W3, b3, W4, b4):
    logits, probs = _constants()
    idx, inv, ids_sorted = _selection(inputs)
    inv2d = inv.reshape(-1, SC_CHUNK)
    xpad = jnp.pad(inputs, ((0, 0), (0, ROWW - INF)))
    x_sorted = _sc_row_scatter(xpad, inv2d)
    ids3 = ids_sorted.reshape(GRID, 1, NSUB * TILE)
    y_flat = _grouped_mlp(ids_sorted, ids3, x_sorted,
                          (W0, W1, W2, W3, W4), (b0, b1, b2, b3, b4))
    out_flat = _sc_row_gather(y_flat, inv2d)
    model_outputs = out_flat[:, :NPASS * OUTF].reshape(
        BATCH, NPASS, OUTF).transpose(1, 0, 2)
    return (model_outputs, idx, logits, probs)


# speedup vs baseline: 1553.1196x; 1.0807x over previous
import functools

import jax
import jax.numpy as jnp
import numpy as np
from jax.experimental import pallas as pl
from jax.experimental.pallas import tpu as pltpu
from jax.experimental.pallas import tpu_sc as plsc

NUM_MODELS = 64
BATCH = 8192
HID = 64
INF = 6
OUTF = 3
NPASS = 4
ROWW = 16
TILE = 128
NSUB = 4
GRID = BATCH // (TILE * NSUB)
SEL_TILE = 1024
SC_WORKERS = 32
SC_CHUNK = 128


RANK_CHUNK = 512


def _sel_rank_kernel(xt_ref, idx_ref, inv_ref, ids_ref, logit_ref,
                     prob_ref):
    logit_ref[...] = jnp.ones_like(logit_ref)
    prob_ref[...] = jnp.full_like(prob_ref, 1.0 / NUM_MODELS)
    xt = xt_ref[...]
    x0 = xt[0:1, :]
    x2 = xt[2:3, :]
    ang = jnp.arctan2(x2, x0)
    ang = jnp.mod(ang + 2 * np.pi, 2 * np.pi) / (2 * np.pi) * NUM_MODELS
    idxi = jnp.floor(ang).astype(jnp.int32)
    idx_ref[...] = idxi

    erow = jax.lax.broadcasted_iota(jnp.int32, (NUM_MODELS, BATCH), 0)
    onehot = (idxi == erow).astype(jnp.float32)
    onehot_b = onehot.astype(jnp.bfloat16)
    total = jnp.sum(onehot, axis=1, keepdims=True)
    lr = jax.lax.broadcasted_iota(jnp.int32, (NUM_MODELS, NUM_MODELS), 0)
    lc = jax.lax.broadcasted_iota(jnp.int32, (NUM_MODELS, NUM_MODELS), 1)
    lstrict = (lc < lr).astype(jnp.float32)
    lincl = (lc <= lr).astype(jnp.float32)
    goff = jax.lax.dot_general(lstrict, total, (((1,), (0,)), ((), ())),
                               preferred_element_type=jnp.float32)
    cum = jax.lax.dot_general(lincl, total, (((1,), (0,)), ((), ())),
                              preferred_element_type=jnp.float32)

    posi = jax.lax.broadcasted_iota(jnp.int32, (NUM_MODELS, BATCH), 1)
    ids_row = jnp.sum(
        (cum.astype(jnp.int32) <= posi).astype(jnp.int32),
        axis=0, keepdims=True)
    for i in range(GRID):
        seg = NSUB * TILE
        ids_ref[i, :, :] = ids_row[:, i * seg:(i + 1) * seg]

    tr = jax.lax.broadcasted_iota(jnp.int32, (RANK_CHUNK, RANK_CHUNK), 0)
    tc = jax.lax.broadcasted_iota(jnp.int32, (RANK_CHUNK, RANK_CHUNK), 1)
    tupper = (tr < tc).astype(jnp.bfloat16)
    carry = jnp.zeros((NUM_MODELS, 1), jnp.float32)
    for i in range(BATCH // RANK_CHUNK):
        oc = onehot[:, i * RANK_CHUNK:(i + 1) * RANK_CHUNK]
        ocb = onehot_b[:, i * RANK_CHUNK:(i + 1) * RANK_CHUNK]
        wt = jax.lax.dot_general(ocb, tupper, (((1,), (0,)), ((), ())),
                                 preferred_element_type=jnp.float32)
        pos = jnp.sum(oc * (wt + carry + goff), axis=0, keepdims=True)
        nsc = RANK_CHUNK // SC_CHUNK
        inv_ref[pl.ds(i * nsc, nsc), :] = pos.astype(jnp.int32).reshape(
            nsc, SC_CHUNK)
        carry = carry + jnp.sum(oc, axis=1, keepdims=True)


def _selection(inputs):
    xt = inputs.T
    idx, inv, ids_sorted, logits, probs = pl.pallas_call(
        _sel_rank_kernel,
        in_specs=[pl.BlockSpec((INF, BATCH), lambda: (0, 0))],
        out_specs=[
            pl.BlockSpec((1, BATCH), lambda: (0, 0)),
            pl.BlockSpec((BATCH // SC_CHUNK, SC_CHUNK), lambda: (0, 0)),
            pl.BlockSpec((GRID, 1, NSUB * TILE), lambda: (0, 0, 0)),
            pl.BlockSpec((BATCH, NUM_MODELS), lambda: (0, 0)),
            pl.BlockSpec((BATCH, NUM_MODELS), lambda: (0, 0)),
        ],
        out_shape=[
            jax.ShapeDtypeStruct((1, BATCH), jnp.int32),
            jax.ShapeDtypeStruct((BATCH // SC_CHUNK, SC_CHUNK), jnp.int32),
            jax.ShapeDtypeStruct((GRID, 1, NSUB * TILE), jnp.int32),
            jax.ShapeDtypeStruct((BATCH, NUM_MODELS), jnp.float32),
            jax.ShapeDtypeStruct((BATCH, NUM_MODELS), jnp.float32),
        ],
    )(xt)
    return idx.reshape(BATCH), inv, ids_sorted, logits, probs


def _sc_row_gather(table, idx2d):
    nrows, ncols = table.shape
    b_per_w = nrows // SC_WORKERS
    nchunk = b_per_w // SC_CHUNK
    mesh = plsc.VectorSubcoreMesh(core_axis_name="c", subcore_axis_name="s")

    @functools.partial(
        pl.kernel, mesh=mesh,
        out_type=jax.ShapeDtypeStruct((nrows, ncols), jnp.float32),
        compiler_params=pltpu.CompilerParams(use_tc_tiling_on_sc=False),
        scratch_types=[
            pltpu.VMEM((nchunk, SC_CHUNK), jnp.int32),
            pltpu.VMEM((b_per_w, ncols), jnp.float32),
            pltpu.SemaphoreType.DMA,
        ],
    )
    def k(table_hbm, idx_hbm, out_hbm, idx_v, rows_v, sem):
        wid = jax.lax.axis_index("s") * 2 + jax.lax.axis_index("c")
        base = wid * b_per_w
        pltpu.sync_copy(idx_hbm.at[pl.ds(wid * nchunk, nchunk)], idx_v)
        copies = [
            pltpu.async_copy(table_hbm.at[idx_v.at[j]],
                             rows_v.at[pl.ds(j * SC_CHUNK, SC_CHUNK)], sem)
            for j in range(nchunk)
        ]
        for c in copies:
            c.wait()
        pltpu.sync_copy(rows_v, out_hbm.at[pl.ds(base, b_per_w)])

    return k(table, idx2d)


def _sc_row_scatter(rows, idx2d):
    nrows, ncols = rows.shape
    b_per_w = nrows // SC_WORKERS
    nchunk = b_per_w // SC_CHUNK
    mesh = plsc.VectorSubcoreMesh(core_axis_name="c", subcore_axis_name="s")

    @functools.partial(
        pl.kernel, mesh=mesh,
        out_type=jax.ShapeDtypeStruct((nrows, ncols), jnp.float32),
        compiler_params=pltpu.CompilerParams(use_tc_tiling_on_sc=False),
        scratch_types=[
            pltpu.VMEM((nchunk, SC_CHUNK), jnp.int32),
            pltpu.VMEM((b_per_w, ncols), jnp.float32),
            pltpu.SemaphoreType.DMA,
        ],
    )
    def k(rows_hbm, idx_hbm, out_hbm, idx_v, rows_v, sem):
        wid = jax.lax.axis_index("s") * 2 + jax.lax.axis_index("c")
        base = wid * b_per_w
        pltpu.sync_copy(idx_hbm.at[pl.ds(wid * nchunk, nchunk)], idx_v)
        pltpu.sync_copy(rows_hbm.at[pl.ds(base, b_per_w)], rows_v)
        copies = [
            pltpu.async_copy(rows_v.at[pl.ds(j * SC_CHUNK, SC_CHUNK)],
                             out_hbm.at[idx_v.at[j]], sem)
            for j in range(nchunk)
        ]
        for c in copies:
            c.wait()

    return k(rows, idx2d)


def _mlp_kernel(ids_sm, ids3_ref, x_ref, w0, b0, w1, b1, w2, b2,
                w3, b3, w4, b4, out_ref):
    g = pl.program_id(0)
    xin = x_ref[...]

    lane = jax.lax.broadcasted_iota(jnp.int32, (1, NPASS * HID), 1)
    pmask = ((lane % HID) <
             (HID // NPASS) * (lane // HID + 1)).astype(jnp.float32)

    zhh = jnp.zeros((HID, HID), jnp.float32)
    zoh = jnp.zeros((OUTF, HID), jnp.float32)

    def bdiag(w, zero):
        rows = []
        for j in range(NPASS):
            blocks = [zero] * NPASS
            blocks[j] = w
            rows.append(jnp.concatenate(blocks, axis=1))
        return jnp.concatenate(rows, axis=0)

    x6s, idcs, los, his, spans = [], [], [], [], []
    for s in range(NSUB):
        base = (g * NSUB + s) * TILE
        lo = ids_sm[base]
        hi = ids_sm[base + TILE - 1]
        xs = xin[s * TILE:(s + 1) * TILE]
        x6s.append(xs[:, :INF])
        idrow = ids3_ref[0, 0:1, s * TILE:(s + 1) * TILE]
        idcs.append(jnp.transpose(idrow, (1, 0)))
        los.append(lo)
        his.append(hi)
        spans.append((hi - lo + 2) // 2)
    nmax = spans[0]
    for s in range(1, NSUB):
        nmax = jnp.maximum(nmax, spans[s])

    def body(i, accs):
        chains = []
        for s in range(NSUB):
            ea = jnp.minimum(los[s] + 2 * i, his[s])
            eb = jnp.minimum(ea + 1, his[s])
            chains.append((s, ea))
            chains.append((s, eb))
        xs_l = []
        for s, e in chains:
            w0c = jnp.concatenate([w0[e]] * NPASS, axis=0)
            b0c = jnp.concatenate([b0[e]] * NPASS, axis=1)
            y = jax.lax.dot_general(x6s[s], w0c, (((1,), (1,)), ((), ())),
                                    preferred_element_type=jnp.float32)
            xs_l.append(jnp.maximum(y + b0c, 0.0) * pmask)
        for w, b in ((w1, b1), (w2, b2), (w3, b3)):
            nxt = []
            for (s, e), xs in zip(chains, xs_l):
                wb = bdiag(w[e], zhh)
                bc = jnp.concatenate([b[e]] * NPASS, axis=1)
                y = jax.lax.dot_general(xs, wb, (((1,), (1,)), ((), ())),
                                        preferred_element_type=jnp.float32)
                nxt.append(jnp.maximum(y + bc, 0.0) * pmask)
            xs_l = nxt
        out = list(accs)
        for (s, e), xs in zip(chains, xs_l):
            w4c = bdiag(w4[e], zoh)
            b4c = jnp.concatenate([b4[e]] * NPASS, axis=1)
            y = jax.lax.dot_general(xs, w4c, (((1,), (1,)), ((), ())),
                                    preferred_element_type=jnp.float32) + b4c
            out[s] = jnp.where(idcs[s] == e, y, out[s])
        return tuple(out)

    accs = tuple(jnp.zeros((TILE, NPASS * OUTF), jnp.float32)
                 for _ in range(NSUB))
    accs = jax.lax.fori_loop(0, nmax, body, accs)
    zpad = jnp.zeros((TILE, ROWW - NPASS * OUTF), jnp.float32)
    rows = [jnp.concatenate([accs[s], zpad], axis=1) for s in range(NSUB)]
    out_ref[...] = jnp.concatenate(rows, axis=0)


def _full(shape):
    zeros = tuple(0 for _ in shape)
    return pl.BlockSpec(shape, lambda g, ids, z=zeros: z)


def _grouped_mlp(ids_sorted, ids3, x_sorted, Ws, bs):
    b0r, b1r, b2r, b3r = (b.reshape(NUM_MODELS, 1, HID) for b in bs[:4])
    b4r = bs[4].reshape(NUM_MODELS, 1, OUTF)
    grid_spec = pltpu.PrefetchScalarGridSpec(
        num_scalar_prefetch=1,
        grid=(GRID,),
        in_specs=[
            pl.BlockSpec((1, 1, NSUB * TILE), lambda g, ids: (g, 0, 0)),
            pl.BlockSpec((NSUB * TILE, ROWW), lambda g, ids: (g, 0)),
            _full((NUM_MODELS, HID, INF)),
            _full((NUM_MODELS, 1, HID)),
            _full((NUM_MODELS, HID, HID)),
            _full((NUM_MODELS, 1, HID)),
            _full((NUM_MODELS, HID, HID)),
            _full((NUM_MODELS, 1, HID)),
            _full((NUM_MODELS, HID, HID)),
            _full((NUM_MODELS, 1, HID)),
            _full((NUM_MODELS, OUTF, HID)),
            _full((NUM_MODELS, 1, OUTF)),
        ],
        out_specs=pl.BlockSpec((NSUB * TILE, ROWW), lambda g, ids: (g, 0)),
    )
    return pl.pallas_call(
        _mlp_kernel,
        grid_spec=grid_spec,
        out_shape=jax.ShapeDtypeStruct((BATCH, ROWW), jnp.float32),
    )(ids_sorted, ids3, x_sorted, Ws[0], b0r, Ws[1], b1r, Ws[2],
      b2r, Ws[3], b3r, Ws[4], b4r)


def kernel(inputs, W0, b0, W1, b1, W2, b2, W3, b3, W4, b4):
    idx, inv, ids_sorted, logits, probs = _selection(inputs)
    xpad = jnp.pad(inputs, ((0, 0), (0, ROWW - INF)))
    x_sorted = _sc_row_scatter(xpad, inv)
    y_flat = _grouped_mlp(ids_sorted.reshape(BATCH), ids_sorted, x_sorted,
                          (W0, W1, W2, W3, W4), (b0, b1, b2, b3, b4))
    out_flat = _sc_row_gather(y_flat, inv)
    model_outputs = out_flat[:, :NPASS * OUTF].reshape(
        BATCH, NPASS, OUTF).transpose(1, 0, 2)
    return (model_outputs, idx, logits, probs)


# speedup vs baseline: 1556.4506x; 1.0021x over previous
import functools

import jax
import jax.numpy as jnp
import numpy as np
from jax.experimental import pallas as pl
from jax.experimental.pallas import tpu as pltpu
from jax.experimental.pallas import tpu_sc as plsc

NUM_MODELS = 64
BATCH = 8192
HID = 64
INF = 6
OUTF = 3
NPASS = 4
ROWW = 16
TILE = 128
NSUB = 8
GRID = BATCH // (TILE * NSUB)
SEL_TILE = 1024
SC_WORKERS = 32
SC_CHUNK = 128


RANK_CHUNK = 512


def _sel_rank_kernel(xt_ref, idx_ref, inv_ref, ids_ref, logit_ref,
                     prob_ref):
    logit_ref[...] = jnp.ones_like(logit_ref)
    prob_ref[...] = jnp.full_like(prob_ref, 1.0 / NUM_MODELS)
    xt = xt_ref[...]
    x0 = xt[0:1, :]
    x2 = xt[2:3, :]
    ang = jnp.arctan2(x2, x0)
    ang = jnp.mod(ang + 2 * np.pi, 2 * np.pi) / (2 * np.pi) * NUM_MODELS
    idxi = jnp.floor(ang).astype(jnp.int32)
    idx_ref[...] = idxi

    erow = jax.lax.broadcasted_iota(jnp.int32, (NUM_MODELS, BATCH), 0)
    onehot = (idxi == erow).astype(jnp.float32)
    onehot_b = onehot.astype(jnp.bfloat16)
    total = jnp.sum(onehot, axis=1, keepdims=True)
    lr = jax.lax.broadcasted_iota(jnp.int32, (NUM_MODELS, NUM_MODELS), 0)
    lc = jax.lax.broadcasted_iota(jnp.int32, (NUM_MODELS, NUM_MODELS), 1)
    lstrict = (lc < lr).astype(jnp.float32)
    lincl = (lc <= lr).astype(jnp.float32)
    goff = jax.lax.dot_general(lstrict, total, (((1,), (0,)), ((), ())),
                               preferred_element_type=jnp.float32)
    cum = jax.lax.dot_general(lincl, total, (((1,), (0,)), ((), ())),
                              preferred_element_type=jnp.float32)

    posi = jax.lax.broadcasted_iota(jnp.int32, (NUM_MODELS, BATCH), 1)
    ids_row = jnp.sum(
        (cum.astype(jnp.int32) <= posi).astype(jnp.int32),
        axis=0, keepdims=True)
    for i in range(GRID):
        seg = NSUB * TILE
        ids_ref[i, :, :] = ids_row[:, i * seg:(i + 1) * seg]

    tr = jax.lax.broadcasted_iota(jnp.int32, (RANK_CHUNK, RANK_CHUNK), 0)
    tc = jax.lax.broadcasted_iota(jnp.int32, (RANK_CHUNK, RANK_CHUNK), 1)
    tupper = (tr < tc).astype(jnp.bfloat16)
    carry = jnp.zeros((NUM_MODELS, 1), jnp.float32)
    for i in range(BATCH // RANK_CHUNK):
        oc = onehot[:, i * RANK_CHUNK:(i + 1) * RANK_CHUNK]
        ocb = onehot_b[:, i * RANK_CHUNK:(i + 1) * RANK_CHUNK]
        wt = jax.lax.dot_general(ocb, tupper, (((1,), (0,)), ((), ())),
                                 preferred_element_type=jnp.float32)
        pos = jnp.sum(oc * (wt + carry + goff), axis=0, keepdims=True)
        nsc = RANK_CHUNK // SC_CHUNK
        inv_ref[pl.ds(i * nsc, nsc), :] = pos.astype(jnp.int32).reshape(
            nsc, SC_CHUNK)
        carry = carry + jnp.sum(oc, axis=1, keepdims=True)


def _selection(inputs):
    xt = inputs.T
    idx, inv, ids_sorted, logits, probs = pl.pallas_call(
        _sel_rank_kernel,
        in_specs=[pl.BlockSpec((INF, BATCH), lambda: (0, 0))],
        out_specs=[
            pl.BlockSpec((1, BATCH), lambda: (0, 0)),
            pl.BlockSpec((BATCH // SC_CHUNK, SC_CHUNK), lambda: (0, 0)),
            pl.BlockSpec((GRID, 1, NSUB * TILE), lambda: (0, 0, 0)),
            pl.BlockSpec((BATCH, NUM_MODELS), lambda: (0, 0)),
            pl.BlockSpec((BATCH, NUM_MODELS), lambda: (0, 0)),
        ],
        out_shape=[
            jax.ShapeDtypeStruct((1, BATCH), jnp.int32),
            jax.ShapeDtypeStruct((BATCH // SC_CHUNK, SC_CHUNK), jnp.int32),
            jax.ShapeDtypeStruct((GRID, 1, NSUB * TILE), jnp.int32),
            jax.ShapeDtypeStruct((BATCH, NUM_MODELS), jnp.float32),
            jax.ShapeDtypeStruct((BATCH, NUM_MODELS), jnp.float32),
        ],
    )(xt)
    return idx.reshape(BATCH), inv, ids_sorted, logits, probs


def _sc_row_gather(table, idx2d):
    nrows, ncols = table.shape
    b_per_w = nrows // SC_WORKERS
    nchunk = b_per_w // SC_CHUNK
    mesh = plsc.VectorSubcoreMesh(core_axis_name="c", subcore_axis_name="s")

    @functools.partial(
        pl.kernel, mesh=mesh,
        out_type=jax.ShapeDtypeStruct((nrows, ncols), jnp.float32),
        compiler_params=pltpu.CompilerParams(use_tc_tiling_on_sc=False),
        scratch_types=[
            pltpu.VMEM((nchunk, SC_CHUNK), jnp.int32),
            pltpu.VMEM((b_per_w, ncols), jnp.float32),
            pltpu.SemaphoreType.DMA,
        ],
    )
    def k(table_hbm, idx_hbm, out_hbm, idx_v, rows_v, sem):
        wid = jax.lax.axis_index("s") * 2 + jax.lax.axis_index("c")
        base = wid * b_per_w
        pltpu.sync_copy(idx_hbm.at[pl.ds(wid * nchunk, nchunk)], idx_v)
        copies = [
            pltpu.async_copy(table_hbm.at[idx_v.at[j]],
                             rows_v.at[pl.ds(j * SC_CHUNK, SC_CHUNK)], sem)
            for j in range(nchunk)
        ]
        for c in copies:
            c.wait()
        pltpu.sync_copy(rows_v, out_hbm.at[pl.ds(base, b_per_w)])

    return k(table, idx2d)


def _sc_row_scatter(rows, idx2d):
    nrows, ncols = rows.shape
    b_per_w = nrows // SC_WORKERS
    nchunk = b_per_w // SC_CHUNK
    mesh = plsc.VectorSubcoreMesh(core_axis_name="c", subcore_axis_name="s")

    @functools.partial(
        pl.kernel, mesh=mesh,
        out_type=jax.ShapeDtypeStruct((nrows, ncols), jnp.float32),
        compiler_params=pltpu.CompilerParams(use_tc_tiling_on_sc=False),
        scratch_types=[
            pltpu.VMEM((nchunk, SC_CHUNK), jnp.int32),
            pltpu.VMEM((b_per_w, ncols), jnp.float32),
            pltpu.SemaphoreType.DMA,
        ],
    )
    def k(rows_hbm, idx_hbm, out_hbm, idx_v, rows_v, sem):
        wid = jax.lax.axis_index("s") * 2 + jax.lax.axis_index("c")
        base = wid * b_per_w
        pltpu.sync_copy(idx_hbm.at[pl.ds(wid * nchunk, nchunk)], idx_v)
        pltpu.sync_copy(rows_hbm.at[pl.ds(base, b_per_w)], rows_v)
        copies = [
            pltpu.async_copy(rows_v.at[pl.ds(j * SC_CHUNK, SC_CHUNK)],
                             out_hbm.at[idx_v.at[j]], sem)
            for j in range(nchunk)
        ]
        for c in copies:
            c.wait()

    return k(rows, idx2d)


def _mlp_kernel(ids_sm, ids3_ref, x_ref, w0, b0, w1, b1, w2, b2,
                w3, b3, w4, b4, out_ref):
    g = pl.program_id(0)
    xin = x_ref[...]

    lane = jax.lax.broadcasted_iota(jnp.int32, (1, NPASS * HID), 1)
    pmask = ((lane % HID) <
             (HID // NPASS) * (lane // HID + 1)).astype(jnp.float32)

    zhh = jnp.zeros((HID, HID), jnp.float32)
    zoh = jnp.zeros((OUTF, HID), jnp.float32)

    def bdiag(w, zero):
        rows = []
        for j in range(NPASS):
            blocks = [zero] * NPASS
            blocks[j] = w
            rows.append(jnp.concatenate(blocks, axis=1))
        return jnp.concatenate(rows, axis=0)

    x6s, idcs, los, his, spans = [], [], [], [], []
    for s in range(NSUB):
        base = (g * NSUB + s) * TILE
        lo = ids_sm[base]
        hi = ids_sm[base + TILE - 1]
        xs = xin[s * TILE:(s + 1) * TILE]
        x6s.append(xs[:, :INF])
        idrow = ids3_ref[0, 0:1, s * TILE:(s + 1) * TILE]
        idcs.append(jnp.transpose(idrow, (1, 0)))
        los.append(lo)
        his.append(hi)
        spans.append((hi - lo + 2) // 2)
    nmax = spans[0]
    for s in range(1, NSUB):
        nmax = jnp.maximum(nmax, spans[s])

    def body(i, accs):
        chains = []
        for s in range(NSUB):
            ea = jnp.minimum(los[s] + 2 * i, his[s])
            eb = jnp.minimum(ea + 1, his[s])
            chains.append((s, ea))
            chains.append((s, eb))
        xs_l = []
        for s, e in chains:
            w0c = jnp.concatenate([w0[e]] * NPASS, axis=0)
            b0c = jnp.concatenate([b0[e]] * NPASS, axis=1)
            y = jax.lax.dot_general(x6s[s], w0c, (((1,), (1,)), ((), ())),
                                    preferred_element_type=jnp.float32)
            xs_l.append(jnp.maximum(y + b0c, 0.0) * pmask)
        for w, b in ((w1, b1), (w2, b2), (w3, b3)):
            nxt = []
            for (s, e), xs in zip(chains, xs_l):
                wb = bdiag(w[e], zhh)
                bc = jnp.concatenate([b[e]] * NPASS, axis=1)
                y = jax.lax.dot_general(xs, wb, (((1,), (1,)), ((), ())),
                                        preferred_element_type=jnp.float32)
                nxt.append(jnp.maximum(y + bc, 0.0) * pmask)
            xs_l = nxt
        out = list(accs)
        for (s, e), xs in zip(chains, xs_l):
            w4c = bdiag(w4[e], zoh)
            b4c = jnp.concatenate([b4[e]] * NPASS, axis=1)
            y = jax.lax.dot_general(xs, w4c, (((1,), (1,)), ((), ())),
                                    preferred_element_type=jnp.float32) + b4c
            out[s] = jnp.where(idcs[s] == e, y, out[s])
        return tuple(out)

    accs = tuple(jnp.zeros((TILE, NPASS * OUTF), jnp.float32)
                 for _ in range(NSUB))
    accs = jax.lax.fori_loop(0, nmax, body, accs)
    zpad = jnp.zeros((TILE, ROWW - NPASS * OUTF), jnp.float32)
    rows = [jnp.concatenate([accs[s], zpad], axis=1) for s in range(NSUB)]
    out_ref[...] = jnp.concatenate(rows, axis=0)


def _full(shape):
    zeros = tuple(0 for _ in shape)
    return pl.BlockSpec(shape, lambda g, ids, z=zeros: z)


def _grouped_mlp(ids_sorted, ids3, x_sorted, Ws, bs):
    b0r, b1r, b2r, b3r = (b.reshape(NUM_MODELS, 1, HID) for b in bs[:4])
    b4r = bs[4].reshape(NUM_MODELS, 1, OUTF)
    grid_spec = pltpu.PrefetchScalarGridSpec(
        num_scalar_prefetch=1,
        grid=(GRID,),
        in_specs=[
            pl.BlockSpec((1, 1, NSUB * TILE), lambda g, ids: (g, 0, 0)),
            pl.BlockSpec((NSUB * TILE, ROWW), lambda g, ids: (g, 0)),
            _full((NUM_MODELS, HID, INF)),
            _full((NUM_MODELS, 1, HID)),
            _full((NUM_MODELS, HID, HID)),
            _full((NUM_MODELS, 1, HID)),
            _full((NUM_MODELS, HID, HID)),
            _full((NUM_MODELS, 1, HID)),
            _full((NUM_MODELS, HID, HID)),
            _full((NUM_MODELS, 1, HID)),
            _full((NUM_MODELS, OUTF, HID)),
            _full((NUM_MODELS, 1, OUTF)),
        ],
        out_specs=pl.BlockSpec((NSUB * TILE, ROWW), lambda g, ids: (g, 0)),
    )
    return pl.pallas_call(
        _mlp_kernel,
        grid_spec=grid_spec,
        out_shape=jax.ShapeDtypeStruct((BATCH, ROWW), jnp.float32),
    )(ids_sorted, ids3, x_sorted, Ws[0], b0r, Ws[1], b1r, Ws[2],
      b2r, Ws[3], b3r, Ws[4], b4r)


def kernel(inputs, W0, b0, W1, b1, W2, b2, W3, b3, W4, b4):
    idx, inv, ids_sorted, logits, probs = _selection(inputs)
    xpad = jnp.pad(inputs, ((0, 0), (0, ROWW - INF)))
    x_sorted = _sc_row_scatter(xpad, inv)
    y_flat = _grouped_mlp(ids_sorted.reshape(BATCH), ids_sorted, x_sorted,
                          (W0, W1, W2, W3, W4), (b0, b1, b2, b3, b4))
    out_flat = _sc_row_gather(y_flat, inv)
    model_outputs = out_flat[:, :NPASS * OUTF].reshape(
        BATCH, NPASS, OUTF).transpose(1, 0, 2)
    return (model_outputs, idx, logits, probs)
